# baseline (speedup 1.0000x reference)
import functools
import jax
import jax.numpy as jnp
from jax import lax
from jax.experimental import pallas as pl
from jax.experimental.pallas import tpu as pltpu
from jax.experimental.pallas import tpu_sc as plsc

EMB = 128; CF = 32; VF = 32; UF = 32; EF = 16
NV = 50000; NC = 50000; NG = 64; E = 300000

EPAD = 311296
NPAD = 51200
TPE = EPAD // 16
EB = 128
NBLK = TPE // EB
RB = 128
RPT = NPAD // 16
NCH = 4


def _leaky(x):
    return jnp.where(x > 0, x, 0.1 * x)


def _pre_body(ve_ref, ce_ref, w2v_ref, w2c_ref, tv_ref, tc_ref):
    pv = jax.lax.dot_general(ve_ref[...], w2v_ref[...], (((1,), (0,)), ((), ())),
                             preferred_element_type=jnp.float32)
    pc = jax.lax.dot_general(ce_ref[...], w2c_ref[...], (((1,), (0,)), ((), ())),
                             preferred_element_type=jnp.float32)
    for j in range(NCH):
        tv_ref[j] = pv[:, 32 * j:32 * j + 32]
        tc_ref[j] = pc[:, 32 * j:32 * j + 32]


def _pre_call(ve, ce, w2v, w2c):
    nb = NPAD // 2048
    return pl.pallas_call(
        _pre_body,
        grid=(nb,),
        in_specs=[
            pl.BlockSpec((2048, EMB), lambda i: (i, 0)),
            pl.BlockSpec((2048, EMB), lambda i: (i, 0)),
            pl.BlockSpec((EMB, EMB), lambda i: (0, 0)),
            pl.BlockSpec((EMB, EMB), lambda i: (0, 0)),
        ],
        out_specs=[pl.BlockSpec((NCH, 2048, 32), lambda i: (0, i, 0))] * 2,
        out_shape=[jax.ShapeDtypeStruct((NCH, NPAD, 32), jnp.float32)] * 2,
    )(ve, ce, w2v, w2c)


def _q_body(sv_ref, sc_ref, w1v_ref, w1c_ref, bv_ref, bc_ref, qv_ref, qc_ref):
    qv = jax.lax.dot_general(sv_ref[...], w1v_ref[...], (((1,), (0,)), ((), ())),
                             preferred_element_type=jnp.float32) + bv_ref[...]
    qc = jax.lax.dot_general(sc_ref[...], w1c_ref[...], (((1,), (0,)), ((), ())),
                             preferred_element_type=jnp.float32) + bc_ref[...]
    for j in range(NCH):
        qv_ref[j] = qv[:, 32 * j:32 * j + 32]
        qc_ref[j] = qc[:, 32 * j:32 * j + 32]


def _q_call(sat_v, sat_c, w1v, w1c, bv, bc):
    nb = EPAD // 2048
    return pl.pallas_call(
        _q_body,
        grid=(nb,),
        in_specs=[
            pl.BlockSpec((2048, EF), lambda i: (i, 0)),
            pl.BlockSpec((2048, EF), lambda i: (i, 0)),
            pl.BlockSpec((EF, EMB), lambda i: (0, 0)),
            pl.BlockSpec((EF, EMB), lambda i: (0, 0)),
            pl.BlockSpec((1, EMB), lambda i: (0, 0)),
            pl.BlockSpec((1, EMB), lambda i: (0, 0)),
        ],
        out_specs=[pl.BlockSpec((NCH, 2048, 32), lambda i: (0, i, 0))] * 2,
        out_shape=[jax.ShapeDtypeStruct((NCH, EPAD, 32), jnp.float32)] * 2,
    )(sat_v, sat_c, w1v, w1c, bv, bc)


def _sc_body(src_m, dst_m, qvc, qcv, tvc, tcv,
             hvc, hcv, cnt_m,
             acc, zeros, idxs, idxd, gbuf, qbuf, mbuf, sem):
    cid = lax.axis_index("c")
    t = lax.axis_index("s")

    def zinit(r, carry):
        zeros[r, pl.ds(0, 16)] = jnp.zeros((16,), jnp.float32)
        zeros[r, pl.ds(16, 16)] = jnp.zeros((16,), jnp.float32)
        return carry
    lax.fori_loop(0, RB, zinit, 0)

    def zero_acc():
        def zb(rb, carry):
            r = pl.multiple_of(t * RPT + rb * RB, 8)
            pltpu.sync_copy(zeros.at[...], acc.at[pl.ds(r, RB)])
            return carry
        lax.fori_loop(0, RPT // RB, zb, 0)

    def writeback(outr, buf):
        def wb(rb, carry):
            r = pl.multiple_of(t * RPT + rb * RB, 8)
            pltpu.sync_copy(acc.at[pl.ds(r, RB)], buf)
            pltpu.sync_copy(buf, outr.at[pl.ds(r, RB)])
            return carry
        lax.fori_loop(0, RPT // RB, wb, 0)

    def feat_pass(srcr, dstr, qr, tblr, outr):
        zero_acc()
        plsc.subcore_barrier()

        def blk(b, carry):
            base = pl.multiple_of(t * TPE + b * EB, 8)
            pltpu.sync_copy(srcr.at[pl.ds(base, EB)], idxs)
            pltpu.sync_copy(dstr.at[pl.ds(base, EB)], idxd)
            pltpu.sync_copy(qr.at[pl.ds(base, EB)], qbuf)
            pltpu.async_copy(tblr.at[idxs], gbuf, sem).wait()

            def cmp(r, icarry):
                for h in (0, 16):
                    s = gbuf[r, pl.ds(h, 16)] + qbuf[r, pl.ds(h, 16)]
                    mbuf[r, pl.ds(h, 16)] = (jnp.maximum(s, 0.0)
                                             + 0.1 * jnp.minimum(s, 0.0))
                return icarry
            lax.fori_loop(0, EB, cmp, 0)

            pltpu.sync_copy(mbuf.at[...], acc.at[idxd], add=True)
            return carry
        lax.fori_loop(0, NBLK, blk, 0)
        plsc.subcore_barrier()
        writeback(outr, mbuf.at[...])

    def cnt_pass(dstr, outr):
        zero_acc()

        def oinit(r, carry):
            mbuf[r, pl.ds(0, 16)] = jnp.full((16,), 1.0, jnp.float32)
            mbuf[r, pl.ds(16, 16)] = jnp.full((16,), 1.0, jnp.float32)
            return carry
        lax.fori_loop(0, EB, oinit, 0)
        plsc.subcore_barrier()

        def blk(b, carry):
            base = pl.multiple_of(t * TPE + b * EB, 8)
            pltpu.sync_copy(dstr.at[pl.ds(base, EB)], idxd)
            pltpu.sync_copy(mbuf.at[...], acc.at[idxd], add=True)
            return carry
        lax.fori_loop(0, NBLK, blk, 0)
        plsc.subcore_barrier()
        writeback(outr, gbuf.at[...])

    for p in (0, 1):
        chunk = 2 * cid + p
        feat_pass(src_m.at[0], dst_m.at[0], qvc.at[chunk], tvc.at[chunk],
                  hvc.at[chunk])
        feat_pass(src_m.at[1], dst_m.at[1], qcv.at[chunk], tcv.at[chunk],
                  hcv.at[chunk])
    cnt_pass(dst_m.at[cid], cnt_m.at[cid])


def _sc_call(src_m, dst_m, qvc, qcv, tvc, tcv):
    mesh = plsc.VectorSubcoreMesh(core_axis_name="c", subcore_axis_name="s")
    f = pl.kernel(
        _sc_body,
        out_type=[
            jax.ShapeDtypeStruct((NCH, NPAD, 32), jnp.float32),
            jax.ShapeDtypeStruct((NCH, NPAD, 32), jnp.float32),
            jax.ShapeDtypeStruct((2, NPAD, 32), jnp.float32),
        ],
        mesh=mesh,
        compiler_params=pltpu.CompilerParams(use_tc_tiling_on_sc=False),
        scratch_types=[
            pltpu.VMEM_SHARED((NPAD, 32), jnp.float32),
            pltpu.VMEM((RB, 32), jnp.float32),
            pltpu.VMEM((EB,), jnp.int32),
            pltpu.VMEM((EB,), jnp.int32),
            pltpu.VMEM((EB, 32), jnp.float32),
            pltpu.VMEM((EB, 32), jnp.float32),
            pltpu.VMEM((EB, 32), jnp.float32),
            pltpu.SemaphoreType.DMA,
        ],
    )
    return f(src_m, dst_m, qvc, qcv, tvc, tcv)


def _node_body(feat_ref, h0_ref, h1_ref, h2_ref, h3_ref, cnt_ref, emb_ref,
               gid_ref, ctx_ref, wf_ref, wh_ref, wc_ref, we_ref, b_ref,
               new_ref, agg_ref, gcnt_ref):
    i = pl.program_id(0)
    cnt = jnp.maximum(cnt_ref[0][:, 0:1], 1.0)
    hs = jnp.concatenate(
        [h0_ref[0], h1_ref[0], h2_ref[0], h3_ref[0]], axis=1)
    h = hs / cnt
    dn = (((1,), (0,)), ((), ()))
    x = (jax.lax.dot_general(feat_ref[...], wf_ref[...], dn,
                             preferred_element_type=jnp.float32)
         + jax.lax.dot_general(h, wh_ref[...], dn,
                               preferred_element_type=jnp.float32)
         + jax.lax.dot_general(emb_ref[...], we_ref[...], dn,
                               preferred_element_type=jnp.float32)
         + b_ref[...])
    tctx = jax.lax.dot_general(ctx_ref[...], wc_ref[...], dn,
                               preferred_element_type=jnp.float32)
    gid = gid_ref[0]
    iota = jax.lax.broadcasted_iota(jnp.int32, (NG, gid.shape[1]), 0)
    ohT = (gid == iota).astype(jnp.float32)
    ctx_part = jax.lax.dot_general(ohT, tctx, (((0,), (0,)), ((), ())),
                                   preferred_element_type=jnp.float32)
    new = _leaky(x + ctx_part)
    new_ref[...] = new

    agg = jax.lax.dot_general(ohT, new, (((1,), (0,)), ((), ())),
                              preferred_element_type=jnp.float32)
    gc = jnp.sum(ohT, axis=1, keepdims=True) * jnp.ones((1, EMB), jnp.float32)

    @pl.when(i == 0)
    def _():
        agg_ref[...] = agg
        gcnt_ref[...] = gc

    @pl.when(i != 0)
    def _():
        agg_ref[...] = agg_ref[...] + agg
        gcnt_ref[...] = gcnt_ref[...] + gc


def _node_call(feat, hs, cnt, cnt_idx, emb, gid3d, ctx_emb, wf, wh, wc, we, b):
    B = 2048
    nb = NPAD // B
    hspec = [pl.BlockSpec((1, B, 32), (lambda i, j=j: (j, i, 0)))
             for j in range(NCH)]
    return pl.pallas_call(
        _node_body,
        grid=(nb,),
        in_specs=[
            pl.BlockSpec((B, 32), lambda i: (i, 0)),
            *hspec,
            pl.BlockSpec((1, B, 32), lambda i: (cnt_idx, i, 0)),
            pl.BlockSpec((B, EMB), lambda i: (i, 0)),
            pl.BlockSpec((1, 1, B), lambda i: (i, 0, 0)),
            pl.BlockSpec((NG, EMB), lambda i: (0, 0)),
            pl.BlockSpec((32, EMB), lambda i: (0, 0)),
            pl.BlockSpec((EMB, EMB), lambda i: (0, 0)),
            pl.BlockSpec((EMB, EMB), lambda i: (0, 0)),
            pl.BlockSpec((EMB, EMB), lambda i: (0, 0)),
            pl.BlockSpec((1, EMB), lambda i: (0, 0)),
        ],
        out_specs=[
            pl.BlockSpec((B, EMB), lambda i: (i, 0)),
            pl.BlockSpec((NG, EMB), lambda i: (0, 0)),
            pl.BlockSpec((NG, EMB), lambda i: (0, 0)),
        ],
        out_shape=[
            jax.ShapeDtypeStruct((NPAD, EMB), jnp.float32),
            jax.ShapeDtypeStruct((NG, EMB), jnp.float32),
            jax.ShapeDtypeStruct((NG, EMB), jnp.float32),
        ],
    )(feat, hs, hs, hs, hs, cnt, emb, gid3d, ctx_emb, wf, wh, wc, we, b)


def _ctx_body(uf_ref, cs_ref, cc_ref, vs_ref, vc_ref, ue_ref,
              wuf_ref, wuc_ref, wuv_ref, wue_ref, b_ref, out_ref):
    dn = (((1,), (0,)), ((), ()))
    c_agg = cs_ref[...] / jnp.maximum(cc_ref[...], 1.0)
    v_agg = vs_ref[...] / jnp.maximum(vc_ref[...], 1.0)
    x = (jax.lax.dot_general(uf_ref[...], wuf_ref[...], dn,
                             preferred_element_type=jnp.float32)
         + jax.lax.dot_general(c_agg, wuc_ref[...], dn,
                               preferred_element_type=jnp.float32)
         + jax.lax.dot_general(v_agg, wuv_ref[...], dn,
                               preferred_element_type=jnp.float32)
         + jax.lax.dot_general(ue_ref[...], wue_ref[...], dn,
                               preferred_element_type=jnp.float32)
         + b_ref[...])
    out_ref[...] = _leaky(x)


def _ctx_call(ctx_feat, cs, cc, vs, vc, ctx_emb, wuf, wuc, wuv, wue, b):
    return pl.pallas_call(
        _ctx_body,
        out_shape=jax.ShapeDtypeStruct((NG, EMB), jnp.float32),
    )(ctx_feat, cs, cc, vs, vc, ctx_emb, wuf, wuc, wuv, wue, b)


def kernel(var_feat, clause_feat, ctx_feat, var_emb, clause_emb, ctx_emb,
           edge_vc, edge_sat_vc, edge_cv, edge_sat_cv,
           graph_id_var, graph_id_clause,
           W_mvc, b_mvc, W_mcv, b_mcv, W_cu, b_cu, W_vu, b_vu, W_uu, b_uu):
    f32 = jnp.float32

    def pad_rows(x, n, val=0.0):
        return jnp.concatenate(
            [x, jnp.full((n - x.shape[0],) + x.shape[1:], val, x.dtype)], axis=0)

    ve_p = pad_rows(var_emb.astype(f32), NPAD)
    ce_p = pad_rows(clause_emb.astype(f32), NPAD)
    vf_p = pad_rows(var_feat.astype(f32), NPAD)
    cf_p = pad_rows(clause_feat.astype(f32), NPAD)
    sat_v_p = pad_rows(edge_sat_vc.astype(f32), EPAD)
    sat_c_p = pad_rows(edge_sat_cv.astype(f32), EPAD)

    src_m = jnp.stack([
        pad_rows(edge_vc[0].astype(jnp.int32), EPAD, 0),
        pad_rows(edge_cv[0].astype(jnp.int32), EPAD, 0),
    ])
    dst_m = jnp.stack([
        pad_rows(edge_vc[1].astype(jnp.int32), EPAD, NC),
        pad_rows(edge_cv[1].astype(jnp.int32), EPAD, NV),
    ])

    gid_v = pad_rows(graph_id_var.astype(jnp.int32), NPAD, NG).reshape(NPAD // 2048, 1, 2048)
    gid_c = pad_rows(graph_id_clause.astype(jnp.int32), NPAD, NG).reshape(NPAD // 2048, 1, 2048)

    W_mvc = W_mvc.astype(f32); W_mcv = W_mcv.astype(f32)
    w1v, w2v = W_mvc[:EF], W_mvc[EF:]
    w1c, w2c = W_mcv[:EF], W_mcv[EF:]
    bv = b_mvc.astype(f32).reshape(1, EMB)
    bc = b_mcv.astype(f32).reshape(1, EMB)

    W_cu = W_cu.astype(f32); W_vu = W_vu.astype(f32); W_uu = W_uu.astype(f32)
    wcu_f, wcu_h, wcu_x, wcu_e = W_cu[:CF], W_cu[CF:CF + EMB], W_cu[CF + EMB:CF + 2 * EMB], W_cu[CF + 2 * EMB:]
    wvu_f, wvu_h, wvu_x, wvu_e = W_vu[:VF], W_vu[VF:VF + EMB], W_vu[VF + EMB:VF + 2 * EMB], W_vu[VF + 2 * EMB:]
    wuu_f, wuu_c, wuu_v, wuu_e = W_uu[:UF], W_uu[UF:UF + EMB], W_uu[UF + EMB:UF + 2 * EMB], W_uu[UF + 2 * EMB:]
    bcu = b_cu.astype(f32).reshape(1, EMB)
    bvu = b_vu.astype(f32).reshape(1, EMB)
    buu = b_uu.astype(f32).reshape(1, EMB)
    ctx_emb = ctx_emb.astype(f32)
    ctx_feat = ctx_feat.astype(f32)

    tvc, tcv = _pre_call(ve_p, ce_p, w2v, w2c)
    qvc, qcv = _q_call(sat_v_p, sat_c_p, w1v, w1c, bv, bc)

    hvc, hcv, cnt_m = _sc_call(src_m, dst_m, qvc, qcv, tvc, tcv)

    new_c_p, c_agg_s, c_gcnt = _node_call(
        cf_p, hvc, cnt_m, 0, ce_p, gid_c, ctx_emb,
        wcu_f, wcu_h, wcu_x, wcu_e, bcu)
    new_v_p, v_agg_s, v_gcnt = _node_call(
        vf_p, hcv, cnt_m, 1, ve_p, gid_v, ctx_emb,
        wvu_f, wvu_h, wvu_x, wvu_e, bvu)

    new_u = _ctx_call(ctx_feat, c_agg_s, c_gcnt, v_agg_s, v_gcnt, ctx_emb,
                      wuu_f, wuu_c, wuu_v, wuu_e, buu)

    return (new_v_p[:NV], new_c_p[:NC], new_u)


# speedup vs baseline: 1.4576x; 1.4576x over previous
import functools
import jax
import jax.numpy as jnp
from jax import lax
from jax.experimental import pallas as pl
from jax.experimental.pallas import tpu as pltpu
from jax.experimental.pallas import tpu_sc as plsc

EMB = 128; CF = 32; VF = 32; UF = 32; EF = 16
NV = 50000; NC = 50000; NG = 64; E = 300000

EPAD = 311296
NPAD = 51200
TPE = EPAD // 16
EB = 128
NBLK = TPE // EB
RB = 128
RPT = NPAD // 16
NCH = 4


def _leaky(x):
    return jnp.where(x > 0, x, 0.1 * x)


def _pre_body(ve_ref, ce_ref, w2v_ref, w2c_ref, tv_ref, tc_ref):
    pv = jax.lax.dot_general(ve_ref[...], w2v_ref[...], (((1,), (0,)), ((), ())),
                             preferred_element_type=jnp.float32)
    pc = jax.lax.dot_general(ce_ref[...], w2c_ref[...], (((1,), (0,)), ((), ())),
                             preferred_element_type=jnp.float32)
    for j in range(NCH):
        tv_ref[j] = pv[:, 32 * j:32 * j + 32]
        tc_ref[j] = pc[:, 32 * j:32 * j + 32]


def _pre_call(ve, ce, w2v, w2c):
    nb = NPAD // 2048
    return pl.pallas_call(
        _pre_body,
        grid=(nb,),
        in_specs=[
            pl.BlockSpec((2048, EMB), lambda i: (i, 0)),
            pl.BlockSpec((2048, EMB), lambda i: (i, 0)),
            pl.BlockSpec((EMB, EMB), lambda i: (0, 0)),
            pl.BlockSpec((EMB, EMB), lambda i: (0, 0)),
        ],
        out_specs=[pl.BlockSpec((NCH, 2048, 32), lambda i: (0, i, 0))] * 2,
        out_shape=[jax.ShapeDtypeStruct((NCH, NPAD, 32), jnp.float32)] * 2,
    )(ve, ce, w2v, w2c)


def _q_body(sv_ref, sc_ref, w1v_ref, w1c_ref, bv_ref, bc_ref, qv_ref, qc_ref):
    qv = jax.lax.dot_general(sv_ref[...], w1v_ref[...], (((1,), (0,)), ((), ())),
                             preferred_element_type=jnp.float32) + bv_ref[...]
    qc = jax.lax.dot_general(sc_ref[...], w1c_ref[...], (((1,), (0,)), ((), ())),
                             preferred_element_type=jnp.float32) + bc_ref[...]
    for j in range(NCH):
        qv_ref[j] = qv[:, 32 * j:32 * j + 32]
        qc_ref[j] = qc[:, 32 * j:32 * j + 32]


def _q_call(sat_v, sat_c, w1v, w1c, bv, bc):
    nb = EPAD // 2048
    return pl.pallas_call(
        _q_body,
        grid=(nb,),
        in_specs=[
            pl.BlockSpec((2048, EF), lambda i: (i, 0)),
            pl.BlockSpec((2048, EF), lambda i: (i, 0)),
            pl.BlockSpec((EF, EMB), lambda i: (0, 0)),
            pl.BlockSpec((EF, EMB), lambda i: (0, 0)),
            pl.BlockSpec((1, EMB), lambda i: (0, 0)),
            pl.BlockSpec((1, EMB), lambda i: (0, 0)),
        ],
        out_specs=[pl.BlockSpec((NCH, 2048, 32), lambda i: (0, i, 0))] * 2,
        out_shape=[jax.ShapeDtypeStruct((NCH, EPAD, 32), jnp.float32)] * 2,
    )(sat_v, sat_c, w1v, w1c, bv, bc)


def _sc_body(src_m, dst_m, qvc, qcv, tvc, tcv,
             hvc, hcv, cnt_m,
             acc,
             is0, is1, id0, id1, id2, id3,
             g0, g1, q0, q1, m0, m1,
             gsem, isem, ssem, wsem):
    cid = lax.axis_index("c")
    t = lax.axis_index("s")
    isb = (is0, is1)
    idb = (id0, id1, id2, id3)
    gb = (g0, g1)
    qb = (q0, q1)
    mb = (m0, m1)

    def fill(buf, val):
        def fi(r, carry):
            buf[r, pl.ds(0, 16)] = jnp.full((16,), val, jnp.float32)
            buf[r, pl.ds(16, 16)] = jnp.full((16,), val, jnp.float32)
            return carry
        lax.fori_loop(0, EB, fi, 0)

    def zero_acc(zbuf):
        def zi(rb, carry):
            r = pl.multiple_of(t * RPT + rb * RB, 8)
            pltpu.async_copy(zbuf, acc.at[pl.ds(r, RB)], wsem)
            return carry
        lax.fori_loop(0, RPT // RB, zi, 0)

        def zw(rb, carry):
            pltpu.make_async_copy(zbuf, acc.at[pl.ds(0, RB)], wsem).wait()
            return carry
        lax.fori_loop(0, RPT // RB, zw, 0)

    def writeback(outr):
        def wi(rb, carry):
            r = pl.multiple_of(t * RPT + rb * RB, 8)
            pltpu.async_copy(acc.at[pl.ds(r, RB)], outr.at[pl.ds(r, RB)], wsem)
            return carry
        lax.fori_loop(0, RPT // RB, wi, 0)

        def ww(rb, carry):
            pltpu.make_async_copy(acc.at[pl.ds(0, RB)], outr.at[pl.ds(0, RB)],
                                  wsem).wait()
            return carry
        lax.fori_loop(0, RPT // RB, ww, 0)

    def feat_pass(srcr, dstr, qr, tblr, outr):
        fill(m0, 0.0)
        zero_acc(m0)
        plsc.subcore_barrier()

        def ebase(bidx):
            return pl.multiple_of(t * TPE + bidx * EB, 8)

        def load_idx(bidx, s, ds):
            pltpu.async_copy(srcr.at[pl.ds(ebase(bidx), EB)], isb[s], isem)
            pltpu.async_copy(dstr.at[pl.ds(ebase(bidx), EB)], idb[ds], isem)

        def wait_idx(s, ds):
            pltpu.make_async_copy(srcr.at[pl.ds(0, EB)], isb[s], isem).wait()
            pltpu.make_async_copy(dstr.at[pl.ds(0, EB)], idb[ds], isem).wait()

        def issue_gq(bidx, s):
            pltpu.async_copy(tblr.at[isb[s]], gb[s], gsem)
            pltpu.async_copy(qr.at[pl.ds(ebase(bidx), EB)], qb[s], gsem)

        def wait_gq(s):
            pltpu.make_async_copy(tblr.at[isb[s]], gb[s], gsem).wait()
            pltpu.make_async_copy(qr.at[pl.ds(0, EB)], qb[s], gsem).wait()

        def compute(s):
            def cmp(r, carry):
                for h in (0, 16):
                    v = gb[s][r, pl.ds(h, 16)] + qb[s][r, pl.ds(h, 16)]
                    mb[s][r, pl.ds(h, 16)] = jnp.maximum(v, 0.1 * v)
                return carry
            lax.fori_loop(0, EB, cmp, 0)

        def scatter(s, ds):
            pltpu.async_copy(mb[s], acc.at[idb[ds]], ssem, add=True)

        def wait_scat(s):
            pltpu.make_async_copy(mb[s], acc.at[id0], ssem).wait()

        def step(b, s, ds, first, last):
            wait_gq(s)
            if not first:
                wait_scat(s)
            if not last:
                load_idx(b + 2, s, (ds + 2) % 4)
            compute(s)
            scatter(s, ds)
            if not last:
                wait_idx(s, (ds + 2) % 4)
                issue_gq(b + 2, s)

        load_idx(0, 0, 0)
        load_idx(1, 1, 1)
        wait_idx(0, 0)
        issue_gq(0, 0)
        wait_idx(1, 1)
        issue_gq(1, 1)
        step(0, 0, 0, True, False)
        step(1, 1, 1, True, False)
        step(2, 0, 2, False, False)
        step(3, 1, 3, False, False)

        def quad(qi, carry):
            b = qi * 4
            step(b + 0, 0, 0, False, False)
            step(b + 1, 1, 1, False, False)
            step(b + 2, 0, 2, False, False)
            step(b + 3, 1, 3, False, False)
            return carry
        lax.fori_loop(1, NBLK // 4 - 1, quad, 0)
        bL = NBLK - 4
        step(bL + 0, 0, 0, False, False)
        step(bL + 1, 1, 1, False, False)
        step(bL + 2, 0, 2, False, True)
        step(bL + 3, 1, 3, False, True)
        wait_scat(0)
        wait_scat(1)
        plsc.subcore_barrier()
        writeback(outr)

    def cnt_pass(dstr, outr):
        fill(m1, 0.0)
        zero_acc(m1)
        fill(m0, 1.0)
        plsc.subcore_barrier()

        def ebase(bidx):
            return pl.multiple_of(t * TPE + bidx * EB, 8)

        def load_idx(bidx, ds):
            pltpu.async_copy(dstr.at[pl.ds(ebase(bidx), EB)], idb[ds], isem)

        def wait_idx(ds):
            pltpu.make_async_copy(dstr.at[pl.ds(0, EB)], idb[ds], isem).wait()

        def wait_scat():
            pltpu.make_async_copy(m0, acc.at[id0], ssem).wait()

        def step(b, ds, first, last):
            if not first:
                wait_scat()
            if not last:
                load_idx(b + 1, (ds + 1) % 4)
            pltpu.async_copy(m0, acc.at[idb[ds]], ssem, add=True)
            if not last:
                wait_idx((ds + 1) % 4)

        load_idx(0, 0)
        wait_idx(0)
        step(0, 0, True, False)
        step(1, 1, True, False)
        step(2, 2, True, False)
        step(3, 3, False, False)

        def quad(qi, carry):
            b = qi * 4
            step(b + 0, 0, False, False)
            step(b + 1, 1, False, False)
            step(b + 2, 2, False, False)
            step(b + 3, 3, False, False)
            return carry
        lax.fori_loop(1, NBLK // 4 - 1, quad, 0)
        bL = NBLK - 4
        step(bL + 0, 0, False, False)
        step(bL + 1, 1, False, False)
        step(bL + 2, 2, False, False)
        step(bL + 3, 3, False, True)
        for _ in range(3):
            wait_scat()
        plsc.subcore_barrier()
        writeback(outr)

    for p in (0, 1):
        chunk = 2 * cid + p
        feat_pass(src_m.at[0], dst_m.at[0], qvc.at[chunk], tvc.at[chunk],
                  hvc.at[chunk])
        feat_pass(src_m.at[1], dst_m.at[1], qcv.at[chunk], tcv.at[chunk],
                  hcv.at[chunk])
    cnt_pass(dst_m.at[cid], cnt_m.at[cid])


def _sc_call(src_m, dst_m, qvc, qcv, tvc, tcv):
    mesh = plsc.VectorSubcoreMesh(core_axis_name="c", subcore_axis_name="s")
    f = pl.kernel(
        _sc_body,
        out_type=[
            jax.ShapeDtypeStruct((NCH, NPAD, 32), jnp.float32),
            jax.ShapeDtypeStruct((NCH, NPAD, 32), jnp.float32),
            jax.ShapeDtypeStruct((2, NPAD, 32), jnp.float32),
        ],
        mesh=mesh,
        compiler_params=pltpu.CompilerParams(use_tc_tiling_on_sc=False),
        scratch_types=(
            [pltpu.VMEM_SHARED((NPAD, 32), jnp.float32)]
            + [pltpu.VMEM((EB,), jnp.int32)] * 6
            + [pltpu.VMEM((EB, 32), jnp.float32)] * 6
            + [pltpu.SemaphoreType.DMA] * 4
        ),
    )
    return f(src_m, dst_m, qvc, qcv, tvc, tcv)


def _node_body(feat_ref, h0_ref, h1_ref, h2_ref, h3_ref, cnt_ref, emb_ref,
               gid_ref, ctx_ref, wf_ref, wh_ref, wc_ref, we_ref, b_ref,
               new_ref, agg_ref, gcnt_ref):
    i = pl.program_id(0)
    cnt = jnp.maximum(cnt_ref[0][:, 0:1], 1.0)
    hs = jnp.concatenate(
        [h0_ref[0], h1_ref[0], h2_ref[0], h3_ref[0]], axis=1)
    h = hs / cnt
    dn = (((1,), (0,)), ((), ()))
    x = (jax.lax.dot_general(feat_ref[...], wf_ref[...], dn,
                             preferred_element_type=jnp.float32)
         + jax.lax.dot_general(h, wh_ref[...], dn,
                               preferred_element_type=jnp.float32)
         + jax.lax.dot_general(emb_ref[...], we_ref[...], dn,
                               preferred_element_type=jnp.float32)
         + b_ref[...])
    tctx = jax.lax.dot_general(ctx_ref[...], wc_ref[...], dn,
                               preferred_element_type=jnp.float32)
    gid = gid_ref[0]
    iota = jax.lax.broadcasted_iota(jnp.int32, (NG, gid.shape[1]), 0)
    ohT = (gid == iota).astype(jnp.float32)
    ctx_part = jax.lax.dot_general(ohT, tctx, (((0,), (0,)), ((), ())),
                                   preferred_element_type=jnp.float32)
    new = _leaky(x + ctx_part)
    new_ref[...] = new

    agg = jax.lax.dot_general(ohT, new, (((1,), (0,)), ((), ())),
                              preferred_element_type=jnp.float32)
    gc = jnp.sum(ohT, axis=1, keepdims=True) * jnp.ones((1, EMB), jnp.float32)

    @pl.when(i == 0)
    def _():
        agg_ref[...] = agg
        gcnt_ref[...] = gc

    @pl.when(i != 0)
    def _():
        agg_ref[...] = agg_ref[...] + agg
        gcnt_ref[...] = gcnt_ref[...] + gc


def _node_call(feat, hs, cnt, cnt_idx, emb, gid3d, ctx_emb, wf, wh, wc, we, b):
    B = 2048
    nb = NPAD // B
    hspec = [pl.BlockSpec((1, B, 32), (lambda i, j=j: (j, i, 0)))
             for j in range(NCH)]
    return pl.pallas_call(
        _node_body,
        grid=(nb,),
        in_specs=[
            pl.BlockSpec((B, 32), lambda i: (i, 0)),
            *hspec,
            pl.BlockSpec((1, B, 32), lambda i: (cnt_idx, i, 0)),
            pl.BlockSpec((B, EMB), lambda i: (i, 0)),
            pl.BlockSpec((1, 1, B), lambda i: (i, 0, 0)),
            pl.BlockSpec((NG, EMB), lambda i: (0, 0)),
            pl.BlockSpec((32, EMB), lambda i: (0, 0)),
            pl.BlockSpec((EMB, EMB), lambda i: (0, 0)),
            pl.BlockSpec((EMB, EMB), lambda i: (0, 0)),
            pl.BlockSpec((EMB, EMB), lambda i: (0, 0)),
            pl.BlockSpec((1, EMB), lambda i: (0, 0)),
        ],
        out_specs=[
            pl.BlockSpec((B, EMB), lambda i: (i, 0)),
            pl.BlockSpec((NG, EMB), lambda i: (0, 0)),
            pl.BlockSpec((NG, EMB), lambda i: (0, 0)),
        ],
        out_shape=[
            jax.ShapeDtypeStruct((NPAD, EMB), jnp.float32),
            jax.ShapeDtypeStruct((NG, EMB), jnp.float32),
            jax.ShapeDtypeStruct((NG, EMB), jnp.float32),
        ],
    )(feat, hs, hs, hs, hs, cnt, emb, gid3d, ctx_emb, wf, wh, wc, we, b)


def _ctx_body(uf_ref, cs_ref, cc_ref, vs_ref, vc_ref, ue_ref,
              wuf_ref, wuc_ref, wuv_ref, wue_ref, b_ref, out_ref):
    dn = (((1,), (0,)), ((), ()))
    c_agg = cs_ref[...] / jnp.maximum(cc_ref[...], 1.0)
    v_agg = vs_ref[...] / jnp.maximum(vc_ref[...], 1.0)
    x = (jax.lax.dot_general(uf_ref[...], wuf_ref[...], dn,
                             preferred_element_type=jnp.float32)
         + jax.lax.dot_general(c_agg, wuc_ref[...], dn,
                               preferred_element_type=jnp.float32)
         + jax.lax.dot_general(v_agg, wuv_ref[...], dn,
                               preferred_element_type=jnp.float32)
         + jax.lax.dot_general(ue_ref[...], wue_ref[...], dn,
                               preferred_element_type=jnp.float32)
         + b_ref[...])
    out_ref[...] = _leaky(x)


def _ctx_call(ctx_feat, cs, cc, vs, vc, ctx_emb, wuf, wuc, wuv, wue, b):
    return pl.pallas_call(
        _ctx_body,
        out_shape=jax.ShapeDtypeStruct((NG, EMB), jnp.float32),
    )(ctx_feat, cs, cc, vs, vc, ctx_emb, wuf, wuc, wuv, wue, b)


def kernel(var_feat, clause_feat, ctx_feat, var_emb, clause_emb, ctx_emb,
           edge_vc, edge_sat_vc, edge_cv, edge_sat_cv,
           graph_id_var, graph_id_clause,
           W_mvc, b_mvc, W_mcv, b_mcv, W_cu, b_cu, W_vu, b_vu, W_uu, b_uu):
    f32 = jnp.float32

    def pad_rows(x, n, val=0.0):
        return jnp.concatenate(
            [x, jnp.full((n - x.shape[0],) + x.shape[1:], val, x.dtype)], axis=0)

    ve_p = pad_rows(var_emb.astype(f32), NPAD)
    ce_p = pad_rows(clause_emb.astype(f32), NPAD)
    vf_p = pad_rows(var_feat.astype(f32), NPAD)
    cf_p = pad_rows(clause_feat.astype(f32), NPAD)
    sat_v_p = pad_rows(edge_sat_vc.astype(f32), EPAD)
    sat_c_p = pad_rows(edge_sat_cv.astype(f32), EPAD)

    src_m = jnp.stack([
        pad_rows(edge_vc[0].astype(jnp.int32), EPAD, 0),
        pad_rows(edge_cv[0].astype(jnp.int32), EPAD, 0),
    ])
    dst_m = jnp.stack([
        pad_rows(edge_vc[1].astype(jnp.int32), EPAD, NC),
        pad_rows(edge_cv[1].astype(jnp.int32), EPAD, NV),
    ])

    gid_v = pad_rows(graph_id_var.astype(jnp.int32), NPAD, NG).reshape(NPAD // 2048, 1, 2048)
    gid_c = pad_rows(graph_id_clause.astype(jnp.int32), NPAD, NG).reshape(NPAD // 2048, 1, 2048)

    W_mvc = W_mvc.astype(f32); W_mcv = W_mcv.astype(f32)
    w1v, w2v = W_mvc[:EF], W_mvc[EF:]
    w1c, w2c = W_mcv[:EF], W_mcv[EF:]
    bv = b_mvc.astype(f32).reshape(1, EMB)
    bc = b_mcv.astype(f32).reshape(1, EMB)

    W_cu = W_cu.astype(f32); W_vu = W_vu.astype(f32); W_uu = W_uu.astype(f32)
    wcu_f, wcu_h, wcu_x, wcu_e = W_cu[:CF], W_cu[CF:CF + EMB], W_cu[CF + EMB:CF + 2 * EMB], W_cu[CF + 2 * EMB:]
    wvu_f, wvu_h, wvu_x, wvu_e = W_vu[:VF], W_vu[VF:VF + EMB], W_vu[VF + EMB:VF + 2 * EMB], W_vu[VF + 2 * EMB:]
    wuu_f, wuu_c, wuu_v, wuu_e = W_uu[:UF], W_uu[UF:UF + EMB], W_uu[UF + EMB:UF + 2 * EMB], W_uu[UF + 2 * EMB:]
    bcu = b_cu.astype(f32).reshape(1, EMB)
    bvu = b_vu.astype(f32).reshape(1, EMB)
    buu = b_uu.astype(f32).reshape(1, EMB)
    ctx_emb = ctx_emb.astype(f32)
    ctx_feat = ctx_feat.astype(f32)

    tvc, tcv = _pre_call(ve_p, ce_p, w2v, w2c)
    qvc, qcv = _q_call(sat_v_p, sat_c_p, w1v, w1c, bv, bc)

    hvc, hcv, cnt_m = _sc_call(src_m, dst_m, qvc, qcv, tvc, tcv)

    new_c_p, c_agg_s, c_gcnt = _node_call(
        cf_p, hvc, cnt_m, 0, ce_p, gid_c, ctx_emb,
        wcu_f, wcu_h, wcu_x, wcu_e, bcu)
    new_v_p, v_agg_s, v_gcnt = _node_call(
        vf_p, hcv, cnt_m, 1, ve_p, gid_v, ctx_emb,
        wvu_f, wvu_h, wvu_x, wvu_e, bvu)

    new_u = _ctx_call(ctx_feat, c_agg_s, c_gcnt, v_agg_s, v_gcnt, ctx_emb,
                      wuu_f, wuu_c, wuu_v, wuu_e, buu)

    return (new_v_p[:NV], new_c_p[:NC], new_u)


# speedup vs baseline: 2.0836x; 1.4294x over previous
import functools
import jax
import jax.numpy as jnp
from jax import lax
from jax.experimental import pallas as pl
from jax.experimental.pallas import tpu as pltpu
from jax.experimental.pallas import tpu_sc as plsc

EMB = 128; CF = 32; VF = 32; UF = 32; EF = 16
NV = 50000; NC = 50000; NG = 64; E = 300000

EPAD = 311296
NPAD = 51200
TPE = EPAD // 16
EB = 128
NBLK = TPE // EB
RB = 128
RPT = NPAD // 16
NCH = 4


def _leaky(x):
    return jnp.where(x > 0, x, 0.1 * x)


def _pre_body(ve_ref, ce_ref, w2v_ref, w2c_ref, tv_ref, tc_ref):
    pv = jax.lax.dot_general(ve_ref[...], w2v_ref[...], (((1,), (0,)), ((), ())),
                             preferred_element_type=jnp.float32)
    pc = jax.lax.dot_general(ce_ref[...], w2c_ref[...], (((1,), (0,)), ((), ())),
                             preferred_element_type=jnp.float32)
    for j in range(NCH):
        tv_ref[j] = pv[:, 32 * j:32 * j + 32]
        tc_ref[j] = pc[:, 32 * j:32 * j + 32]


def _pre_call(ve, ce, w2v, w2c):
    nb = NPAD // 2048
    return pl.pallas_call(
        _pre_body,
        grid=(nb,),
        in_specs=[
            pl.BlockSpec((2048, EMB), lambda i: (i, 0)),
            pl.BlockSpec((2048, EMB), lambda i: (i, 0)),
            pl.BlockSpec((EMB, EMB), lambda i: (0, 0)),
            pl.BlockSpec((EMB, EMB), lambda i: (0, 0)),
        ],
        out_specs=[pl.BlockSpec((NCH, 2048, 32), lambda i: (0, i, 0))] * 2,
        out_shape=[jax.ShapeDtypeStruct((NCH, NPAD, 32), jnp.float32)] * 2,
    )(ve, ce, w2v, w2c)


def _q_body(sv_ref, sc_ref, wv_ref, wc_ref, bv_ref, bc_ref, qv_ref, qc_ref):
    qv = jax.lax.dot_general(sv_ref[...], wv_ref[...], (((1,), (0,)), ((), ())),
                             preferred_element_type=jnp.float32) + bv_ref[...]
    qc = jax.lax.dot_general(sc_ref[...], wc_ref[...], (((1,), (0,)), ((), ())),
                             preferred_element_type=jnp.float32) + bc_ref[...]
    for j in range(NCH):
        qv_ref[j] = qv[:, 128 * j:128 * j + 128]
        qc_ref[j] = qc[:, 128 * j:128 * j + 128]


def _q_call(sat4_v, sat4_c, wv, wc, bv, bc):
    nb = EPAD // 2048
    return pl.pallas_call(
        _q_body,
        grid=(nb,),
        in_specs=[
            pl.BlockSpec((512, 4 * EF), lambda i: (i, 0)),
            pl.BlockSpec((512, 4 * EF), lambda i: (i, 0)),
            pl.BlockSpec((4 * EF, 4 * EMB), lambda i: (0, 0)),
            pl.BlockSpec((4 * EF, 4 * EMB), lambda i: (0, 0)),
            pl.BlockSpec((1, 4 * EMB), lambda i: (0, 0)),
            pl.BlockSpec((1, 4 * EMB), lambda i: (0, 0)),
        ],
        out_specs=[pl.BlockSpec((NCH, 512, 128), lambda i: (0, i, 0))] * 2,
        out_shape=[jax.ShapeDtypeStruct((NCH, EPAD // 4, 128), jnp.float32)] * 2,
    )(sat4_v, sat4_c, wv, wc, bv, bc)


def _pack_qw(w1, b):
    blocks = w1.reshape(EF, NCH, 32)
    big = jnp.zeros((4, EF, NCH, 4, 32), w1.dtype)
    for a in range(4):
        big = big.at[a, :, :, a, :].set(blocks)
    bbig = jnp.tile(b.reshape(NCH, 1, 32), (1, 4, 1))
    return big.reshape(4 * EF, 4 * EMB), bbig.reshape(1, 4 * EMB)


def _sc_body(src_m, dst_m, qvc, qcv, tvc, tcv,
             hvc, hcv, cnt_m,
             acc,
             is0, is1, id0, id1, id2, id3,
             g0, g1, q0, q1, m0, m1,
             gsem, isem, ssem, wsem):
    cid = lax.axis_index("c")
    t = lax.axis_index("s")
    isb = (is0, is1)
    idb = (id0, id1, id2, id3)
    gb = (g0, g1)
    qb = (q0, q1)
    mb = (m0, m1)

    def fill(buf, val):
        def fi(r, carry):
            buf[r, pl.ds(0, 16)] = jnp.full((16,), val, jnp.float32)
            buf[r, pl.ds(16, 16)] = jnp.full((16,), val, jnp.float32)
            return carry
        lax.fori_loop(0, EB, fi, 0)

    def zero_acc(zbuf):
        def zi(rb, carry):
            r = pl.multiple_of(t * RPT + rb * RB, 8)
            pltpu.async_copy(zbuf, acc.at[pl.ds(r, RB)], wsem)
            return carry
        lax.fori_loop(0, RPT // RB, zi, 0)

        def zw(rb, carry):
            pltpu.make_async_copy(zbuf, acc.at[pl.ds(0, RB)], wsem).wait()
            return carry
        lax.fori_loop(0, RPT // RB, zw, 0)

    def writeback(outr):
        def wi(rb, carry):
            r = pl.multiple_of(t * RPT + rb * RB, 8)
            pltpu.async_copy(acc.at[pl.ds(r, RB)], outr.at[pl.ds(r, RB)], wsem)
            return carry
        lax.fori_loop(0, RPT // RB, wi, 0)

        def ww(rb, carry):
            pltpu.make_async_copy(acc.at[pl.ds(0, RB)], outr.at[pl.ds(0, RB)],
                                  wsem).wait()
            return carry
        lax.fori_loop(0, RPT // RB, ww, 0)

    def feat_pass(srcr, dstr, qr, tblr, outr):
        fill(m0, 0.0)
        zero_acc(m0)
        plsc.subcore_barrier()

        def ebase(bidx):
            return pl.multiple_of(t * TPE + bidx * EB, 8)

        def ebase4(bidx):
            return pl.multiple_of(t * (TPE // 4) + bidx * (EB // 4), 8)

        def load_idx(bidx, s, ds):
            pltpu.async_copy(srcr.at[pl.ds(ebase(bidx), EB)], isb[s], isem)
            pltpu.async_copy(dstr.at[pl.ds(ebase(bidx), EB)], idb[ds], isem)

        def wait_idx(s, ds):
            pltpu.make_async_copy(srcr.at[pl.ds(0, EB)], isb[s], isem).wait()
            pltpu.make_async_copy(dstr.at[pl.ds(0, EB)], idb[ds], isem).wait()

        def issue_gq(bidx, s):
            pltpu.async_copy(tblr.at[isb[s]], gb[s], gsem)
            pltpu.async_copy(qr.at[pl.ds(ebase4(bidx), EB // 4)], qb[s], gsem)

        def wait_gq(s):
            pltpu.make_async_copy(tblr.at[isb[s]], gb[s], gsem).wait()
            pltpu.make_async_copy(qr.at[pl.ds(0, EB // 4)], qb[s], gsem).wait()

        def compute(s):
            def cmp(rq, carry):
                for e in range(4):
                    r = rq * 4 + e
                    for h in (0, 16):
                        v = (gb[s][r, pl.ds(h, 16)]
                             + qb[s][rq, pl.ds(32 * e + h, 16)])
                        mb[s][r, pl.ds(h, 16)] = jnp.maximum(v, 0.1 * v)
                return carry
            lax.fori_loop(0, EB // 4, cmp, 0)

        def scatter(s, ds):
            pltpu.async_copy(mb[s], acc.at[idb[ds]], ssem, add=True)

        def wait_scat(s):
            pltpu.make_async_copy(mb[s], acc.at[id0], ssem).wait()

        def step(b, s, ds, first, last):
            wait_gq(s)
            if not first:
                wait_scat(s)
            if not last:
                load_idx(b + 2, s, (ds + 2) % 4)
            compute(s)
            scatter(s, ds)
            if not last:
                wait_idx(s, (ds + 2) % 4)
                issue_gq(b + 2, s)

        load_idx(0, 0, 0)
        load_idx(1, 1, 1)
        wait_idx(0, 0)
        issue_gq(0, 0)
        wait_idx(1, 1)
        issue_gq(1, 1)
        step(0, 0, 0, True, False)
        step(1, 1, 1, True, False)
        step(2, 0, 2, False, False)
        step(3, 1, 3, False, False)

        def quad(qi, carry):
            b = qi * 4
            step(b + 0, 0, 0, False, False)
            step(b + 1, 1, 1, False, False)
            step(b + 2, 0, 2, False, False)
            step(b + 3, 1, 3, False, False)
            return carry
        lax.fori_loop(1, NBLK // 4 - 1, quad, 0)
        bL = NBLK - 4
        step(bL + 0, 0, 0, False, False)
        step(bL + 1, 1, 1, False, False)
        step(bL + 2, 0, 2, False, True)
        step(bL + 3, 1, 3, False, True)
        wait_scat(0)
        wait_scat(1)
        plsc.subcore_barrier()
        writeback(outr)

    def cnt_pass(dstr, outr):
        fill(m1, 0.0)
        zero_acc(m1)
        fill(m0, 1.0)
        plsc.subcore_barrier()

        def ebase(bidx):
            return pl.multiple_of(t * TPE + bidx * EB, 8)

        def load_idx(bidx, ds):
            pltpu.async_copy(dstr.at[pl.ds(ebase(bidx), EB)], idb[ds], isem)

        def wait_idx(ds):
            pltpu.make_async_copy(dstr.at[pl.ds(0, EB)], idb[ds], isem).wait()

        def wait_scat():
            pltpu.make_async_copy(m0, acc.at[id0], ssem).wait()

        def step(b, ds, first, last):
            if not first:
                wait_scat()
            if not last:
                load_idx(b + 1, (ds + 1) % 4)
            pltpu.async_copy(m0, acc.at[idb[ds]], ssem, add=True)
            if not last:
                wait_idx((ds + 1) % 4)

        load_idx(0, 0)
        wait_idx(0)
        step(0, 0, True, False)
        step(1, 1, True, False)
        step(2, 2, True, False)
        step(3, 3, False, False)

        def quad(qi, carry):
            b = qi * 4
            step(b + 0, 0, False, False)
            step(b + 1, 1, False, False)
            step(b + 2, 2, False, False)
            step(b + 3, 3, False, False)
            return carry
        lax.fori_loop(1, NBLK // 4 - 1, quad, 0)
        bL = NBLK - 4
        step(bL + 0, 0, False, False)
        step(bL + 1, 1, False, False)
        step(bL + 2, 2, False, False)
        step(bL + 3, 3, False, True)
        for _ in range(3):
            wait_scat()
        plsc.subcore_barrier()
        writeback(outr)

    for p in (0, 1):
        chunk = 2 * cid + p
        feat_pass(src_m.at[0], dst_m.at[0], qvc.at[chunk], tvc.at[chunk],
                  hvc.at[chunk])
        feat_pass(src_m.at[1], dst_m.at[1], qcv.at[chunk], tcv.at[chunk],
                  hcv.at[chunk])
    cnt_pass(dst_m.at[cid], cnt_m.at[cid])


def _sc_call(src_m, dst_m, qvc, qcv, tvc, tcv):
    mesh = plsc.VectorSubcoreMesh(core_axis_name="c", subcore_axis_name="s")
    f = pl.kernel(
        _sc_body,
        out_type=[
            jax.ShapeDtypeStruct((NCH, NPAD, 32), jnp.float32),
            jax.ShapeDtypeStruct((NCH, NPAD, 32), jnp.float32),
            jax.ShapeDtypeStruct((2, NPAD, 32), jnp.float32),
        ],
        mesh=mesh,
        compiler_params=pltpu.CompilerParams(use_tc_tiling_on_sc=False),
        scratch_types=(
            [pltpu.VMEM_SHARED((NPAD, 32), jnp.float32)]
            + [pltpu.VMEM((EB,), jnp.int32)] * 6
            + [pltpu.VMEM((EB, 32), jnp.float32)] * 2
            + [pltpu.VMEM((EB // 4, 128), jnp.float32)] * 2
            + [pltpu.VMEM((EB, 32), jnp.float32)] * 2
            + [pltpu.SemaphoreType.DMA] * 4
        ),
    )
    return f(src_m, dst_m, qvc, qcv, tvc, tcv)


def _node_body(feat_ref, h0_ref, h1_ref, h2_ref, h3_ref, cnt_ref, emb_ref,
               gid_ref, ctx_ref, wf_ref, wh_ref, wc_ref, we_ref, b_ref,
               new_ref, agg_ref, gcnt_ref):
    i = pl.program_id(0)
    cnt = jnp.maximum(cnt_ref[0][:, 0:1], 1.0)
    hs = jnp.concatenate(
        [h0_ref[0], h1_ref[0], h2_ref[0], h3_ref[0]], axis=1)
    h = hs / cnt
    dn = (((1,), (0,)), ((), ()))
    x = (jax.lax.dot_general(feat_ref[...], wf_ref[...], dn,
                             preferred_element_type=jnp.float32)
         + jax.lax.dot_general(h, wh_ref[...], dn,
                               preferred_element_type=jnp.float32)
         + jax.lax.dot_general(emb_ref[...], we_ref[...], dn,
                               preferred_element_type=jnp.float32)
         + b_ref[...])
    tctx = jax.lax.dot_general(ctx_ref[...], wc_ref[...], dn,
                               preferred_element_type=jnp.float32)
    gid = gid_ref[0]
    iota = jax.lax.broadcasted_iota(jnp.int32, (NG, gid.shape[1]), 0)
    ohT = (gid == iota).astype(jnp.float32)
    ctx_part = jax.lax.dot_general(ohT, tctx, (((0,), (0,)), ((), ())),
                                   preferred_element_type=jnp.float32)
    new = _leaky(x + ctx_part)
    new_ref[...] = new

    agg = jax.lax.dot_general(ohT, new, (((1,), (0,)), ((), ())),
                              preferred_element_type=jnp.float32)
    gc = jnp.sum(ohT, axis=1, keepdims=True) * jnp.ones((1, EMB), jnp.float32)

    @pl.when(i == 0)
    def _():
        agg_ref[...] = agg
        gcnt_ref[...] = gc

    @pl.when(i != 0)
    def _():
        agg_ref[...] = agg_ref[...] + agg
        gcnt_ref[...] = gcnt_ref[...] + gc


def _node_call(feat, hs, cnt, cnt_idx, emb, gid3d, ctx_emb, wf, wh, wc, we, b):
    B = 2048
    nb = NPAD // B
    hspec = [pl.BlockSpec((1, B, 32), (lambda i, j=j: (j, i, 0)))
             for j in range(NCH)]
    return pl.pallas_call(
        _node_body,
        grid=(nb,),
        in_specs=[
            pl.BlockSpec((B, 32), lambda i: (i, 0)),
            *hspec,
            pl.BlockSpec((1, B, 32), lambda i: (cnt_idx, i, 0)),
            pl.BlockSpec((B, EMB), lambda i: (i, 0)),
            pl.BlockSpec((1, 1, B), lambda i: (i, 0, 0)),
            pl.BlockSpec((NG, EMB), lambda i: (0, 0)),
            pl.BlockSpec((32, EMB), lambda i: (0, 0)),
            pl.BlockSpec((EMB, EMB), lambda i: (0, 0)),
            pl.BlockSpec((EMB, EMB), lambda i: (0, 0)),
            pl.BlockSpec((EMB, EMB), lambda i: (0, 0)),
            pl.BlockSpec((1, EMB), lambda i: (0, 0)),
        ],
        out_specs=[
            pl.BlockSpec((B, EMB), lambda i: (i, 0)),
            pl.BlockSpec((NG, EMB), lambda i: (0, 0)),
            pl.BlockSpec((NG, EMB), lambda i: (0, 0)),
        ],
        out_shape=[
            jax.ShapeDtypeStruct((NPAD, EMB), jnp.float32),
            jax.ShapeDtypeStruct((NG, EMB), jnp.float32),
            jax.ShapeDtypeStruct((NG, EMB), jnp.float32),
        ],
    )(feat, hs, hs, hs, hs, cnt, emb, gid3d, ctx_emb, wf, wh, wc, we, b)


def _ctx_body(uf_ref, cs_ref, cc_ref, vs_ref, vc_ref, ue_ref,
              wuf_ref, wuc_ref, wuv_ref, wue_ref, b_ref, out_ref):
    dn = (((1,), (0,)), ((), ()))
    c_agg = cs_ref[...] / jnp.maximum(cc_ref[...], 1.0)
    v_agg = vs_ref[...] / jnp.maximum(vc_ref[...], 1.0)
    x = (jax.lax.dot_general(uf_ref[...], wuf_ref[...], dn,
                             preferred_element_type=jnp.float32)
         + jax.lax.dot_general(c_agg, wuc_ref[...], dn,
                               preferred_element_type=jnp.float32)
         + jax.lax.dot_general(v_agg, wuv_ref[...], dn,
                               preferred_element_type=jnp.float32)
         + jax.lax.dot_general(ue_ref[...], wue_ref[...], dn,
                               preferred_element_type=jnp.float32)
         + b_ref[...])
    out_ref[...] = _leaky(x)


def _ctx_call(ctx_feat, cs, cc, vs, vc, ctx_emb, wuf, wuc, wuv, wue, b):
    return pl.pallas_call(
        _ctx_body,
        out_shape=jax.ShapeDtypeStruct((NG, EMB), jnp.float32),
    )(ctx_feat, cs, cc, vs, vc, ctx_emb, wuf, wuc, wuv, wue, b)


def kernel(var_feat, clause_feat, ctx_feat, var_emb, clause_emb, ctx_emb,
           edge_vc, edge_sat_vc, edge_cv, edge_sat_cv,
           graph_id_var, graph_id_clause,
           W_mvc, b_mvc, W_mcv, b_mcv, W_cu, b_cu, W_vu, b_vu, W_uu, b_uu):
    f32 = jnp.float32

    def pad_rows(x, n, val=0.0):
        return jnp.concatenate(
            [x, jnp.full((n - x.shape[0],) + x.shape[1:], val, x.dtype)], axis=0)

    ve_p = pad_rows(var_emb.astype(f32), NPAD)
    ce_p = pad_rows(clause_emb.astype(f32), NPAD)
    vf_p = pad_rows(var_feat.astype(f32), NPAD)
    cf_p = pad_rows(clause_feat.astype(f32), NPAD)
    sat_v_p = pad_rows(edge_sat_vc.astype(f32), EPAD)
    sat_c_p = pad_rows(edge_sat_cv.astype(f32), EPAD)

    src_m = jnp.stack([
        pad_rows(edge_vc[0].astype(jnp.int32), EPAD, 0),
        pad_rows(edge_cv[0].astype(jnp.int32), EPAD, 0),
    ])
    dst_m = jnp.stack([
        pad_rows(edge_vc[1].astype(jnp.int32), EPAD, NC),
        pad_rows(edge_cv[1].astype(jnp.int32), EPAD, NV),
    ])

    gid_v = pad_rows(graph_id_var.astype(jnp.int32), NPAD, NG).reshape(NPAD // 2048, 1, 2048)
    gid_c = pad_rows(graph_id_clause.astype(jnp.int32), NPAD, NG).reshape(NPAD // 2048, 1, 2048)

    W_mvc = W_mvc.astype(f32); W_mcv = W_mcv.astype(f32)
    w1v, w2v = W_mvc[:EF], W_mvc[EF:]
    w1c, w2c = W_mcv[:EF], W_mcv[EF:]
    wqv, bqv = _pack_qw(w1v, b_mvc.astype(f32))
    wqc, bqc = _pack_qw(w1c, b_mcv.astype(f32))
    sat4_v = sat_v_p.reshape(EPAD // 4, 4 * EF)
    sat4_c = sat_c_p.reshape(EPAD // 4, 4 * EF)

    W_cu = W_cu.astype(f32); W_vu = W_vu.astype(f32); W_uu = W_uu.astype(f32)
    wcu_f, wcu_h, wcu_x, wcu_e = W_cu[:CF], W_cu[CF:CF + EMB], W_cu[CF + EMB:CF + 2 * EMB], W_cu[CF + 2 * EMB:]
    wvu_f, wvu_h, wvu_x, wvu_e = W_vu[:VF], W_vu[VF:VF + EMB], W_vu[VF + EMB:VF + 2 * EMB], W_vu[VF + 2 * EMB:]
    wuu_f, wuu_c, wuu_v, wuu_e = W_uu[:UF], W_uu[UF:UF + EMB], W_uu[UF + EMB:UF + 2 * EMB], W_uu[UF + 2 * EMB:]
    bcu = b_cu.astype(f32).reshape(1, EMB)
    bvu = b_vu.astype(f32).reshape(1, EMB)
    buu = b_uu.astype(f32).reshape(1, EMB)
    ctx_emb = ctx_emb.astype(f32)
    ctx_feat = ctx_feat.astype(f32)

    tvc, tcv = _pre_call(ve_p, ce_p, w2v, w2c)
    qvc, qcv = _q_call(sat4_v, sat4_c, wqv, wqc, bqv, bqc)

    hvc, hcv, cnt_m = _sc_call(src_m, dst_m, qvc, qcv, tvc, tcv)

    new_c_p, c_agg_s, c_gcnt = _node_call(
        cf_p, hvc, cnt_m, 0, ce_p, gid_c, ctx_emb,
        wcu_f, wcu_h, wcu_x, wcu_e, bcu)
    new_v_p, v_agg_s, v_gcnt = _node_call(
        vf_p, hcv, cnt_m, 1, ve_p, gid_v, ctx_emb,
        wvu_f, wvu_h, wvu_x, wvu_e, bvu)

    new_u = _ctx_call(ctx_feat, c_agg_s, c_gcnt, v_agg_s, v_gcnt, ctx_emb,
                      wuu_f, wuu_c, wuu_v, wuu_e, buu)

    return (new_v_p[:NV], new_c_p[:NC], new_u)


# speedup vs baseline: 2.1753x; 1.0440x over previous
import functools
import jax
import jax.numpy as jnp
from jax import lax
from jax.experimental import pallas as pl
from jax.experimental.pallas import tpu as pltpu
from jax.experimental.pallas import tpu_sc as plsc

EMB = 128; CF = 32; VF = 32; UF = 32; EF = 16
NV = 50000; NC = 50000; NG = 64; E = 300000

EPAD = 311296
NPAD = 51200
TPE = EPAD // 16
EB = 128
NBLK = TPE // EB
RB = 128
RPT = NPAD // 16
NCH = 4


def _leaky(x):
    return jnp.where(x > 0, x, 0.1 * x)


def _pre_body(ve_ref, ce_ref, w2v_ref, w2c_ref, tv_ref, tc_ref):
    pv = jax.lax.dot_general(ve_ref[...], w2v_ref[...], (((1,), (0,)), ((), ())),
                             preferred_element_type=jnp.float32)
    pc = jax.lax.dot_general(ce_ref[...], w2c_ref[...], (((1,), (0,)), ((), ())),
                             preferred_element_type=jnp.float32)
    for j in range(NCH):
        tv_ref[j] = pv[:, 128 * j:128 * j + 128]
        tc_ref[j] = pc[:, 128 * j:128 * j + 128]


def _pre_call(ve4, ce4, w2v, w2c):
    nb = NPAD // 2048
    return pl.pallas_call(
        _pre_body,
        grid=(nb,),
        in_specs=[
            pl.BlockSpec((512, 4 * EMB), lambda i: (i, 0)),
            pl.BlockSpec((512, 4 * EMB), lambda i: (i, 0)),
            pl.BlockSpec((4 * EMB, 4 * EMB), lambda i: (0, 0)),
            pl.BlockSpec((4 * EMB, 4 * EMB), lambda i: (0, 0)),
        ],
        out_specs=[pl.BlockSpec((NCH, 512, 128), lambda i: (0, i, 0))] * 2,
        out_shape=[jax.ShapeDtypeStruct((NCH, NPAD // 4, 128), jnp.float32)] * 2,
    )(ve4, ce4, w2v, w2c)


def _pack_tw(w2):
    blocks = w2.reshape(EMB, NCH, 32)
    big = jnp.zeros((4, EMB, NCH, 4, 32), w2.dtype)
    for a in range(4):
        big = big.at[a, :, :, a, :].set(blocks)
    return big.reshape(4 * EMB, 4 * EMB)


def _q_body(sv_ref, sc_ref, wv_ref, wc_ref, bv_ref, bc_ref, qv_ref, qc_ref):
    qv = jax.lax.dot_general(sv_ref[...], wv_ref[...], (((1,), (0,)), ((), ())),
                             preferred_element_type=jnp.float32) + bv_ref[...]
    qc = jax.lax.dot_general(sc_ref[...], wc_ref[...], (((1,), (0,)), ((), ())),
                             preferred_element_type=jnp.float32) + bc_ref[...]
    for j in range(NCH):
        qv_ref[j] = qv[:, 128 * j:128 * j + 128]
        qc_ref[j] = qc[:, 128 * j:128 * j + 128]


def _q_call(sat4_v, sat4_c, wv, wc, bv, bc):
    nb = EPAD // 2048
    return pl.pallas_call(
        _q_body,
        grid=(nb,),
        in_specs=[
            pl.BlockSpec((512, 4 * EF), lambda i: (i, 0)),
            pl.BlockSpec((512, 4 * EF), lambda i: (i, 0)),
            pl.BlockSpec((4 * EF, 4 * EMB), lambda i: (0, 0)),
            pl.BlockSpec((4 * EF, 4 * EMB), lambda i: (0, 0)),
            pl.BlockSpec((1, 4 * EMB), lambda i: (0, 0)),
            pl.BlockSpec((1, 4 * EMB), lambda i: (0, 0)),
        ],
        out_specs=[pl.BlockSpec((NCH, 512, 128), lambda i: (0, i, 0))] * 2,
        out_shape=[jax.ShapeDtypeStruct((NCH, EPAD // 4, 128), jnp.float32)] * 2,
    )(sat4_v, sat4_c, wv, wc, bv, bc)


def _pack_qw(w1, b):
    blocks = w1.reshape(EF, NCH, 32)
    big = jnp.zeros((4, EF, NCH, 4, 32), w1.dtype)
    for a in range(4):
        big = big.at[a, :, :, a, :].set(blocks)
    bbig = jnp.tile(b.reshape(NCH, 1, 32), (1, 4, 1))
    return big.reshape(4 * EF, 4 * EMB), bbig.reshape(1, 4 * EMB)


def _sc_body(src_m, dst_m, qvc, qcv, tvc, tcv,
             hvc, hcv, cnt_m,
             acc,
             is0, is1, id0, id1, id2, id3,
             g0, g1, q0, q1, m0, m1,
             gsem, isem, ssem, wsem):
    cid = lax.axis_index("c")
    t = lax.axis_index("s")
    isb = (is0, is1)
    idb = (id0, id1, id2, id3)
    gb = (g0, g1)
    qb = (q0, q1)
    mb = (m0, m1)

    def fill(buf, val):
        def fi(r, carry):
            buf[r, pl.ds(0, 16)] = jnp.full((16,), val, jnp.float32)
            buf[r, pl.ds(16, 16)] = jnp.full((16,), val, jnp.float32)
            return carry
        lax.fori_loop(0, EB, fi, 0)

    def zero_acc(zbuf):
        def zi(rb, carry):
            r = pl.multiple_of(t * RPT + rb * RB, 8)
            pltpu.async_copy(zbuf, acc.at[pl.ds(r, RB)], wsem)
            return carry
        lax.fori_loop(0, RPT // RB, zi, 0)

        def zw(rb, carry):
            pltpu.make_async_copy(zbuf, acc.at[pl.ds(0, RB)], wsem).wait()
            return carry
        lax.fori_loop(0, RPT // RB, zw, 0)

    def writeback(outr):
        def wi(rb, carry):
            r = pl.multiple_of(t * RPT + rb * RB, 8)
            pltpu.async_copy(acc.at[pl.ds(r, RB)], outr.at[pl.ds(r, RB)], wsem)
            return carry
        lax.fori_loop(0, RPT // RB, wi, 0)

        def ww(rb, carry):
            pltpu.make_async_copy(acc.at[pl.ds(0, RB)], outr.at[pl.ds(0, RB)],
                                  wsem).wait()
            return carry
        lax.fori_loop(0, RPT // RB, ww, 0)

    def feat_pass(srcr, dstr, qr, tblr, outr):
        fill(m0, 0.0)
        zero_acc(m0)
        plsc.subcore_barrier()

        def ebase(bidx):
            return pl.multiple_of(t * TPE + bidx * EB, 8)

        def ebase4(bidx):
            return pl.multiple_of(t * (TPE // 4) + bidx * (EB // 4), 8)

        def load_idx(bidx, s, ds):
            pltpu.async_copy(srcr.at[pl.ds(ebase(bidx), EB)], isb[s], isem)
            pltpu.async_copy(dstr.at[pl.ds(ebase(bidx), EB)], idb[ds], isem)

        def wait_idx(s, ds):
            pltpu.make_async_copy(srcr.at[pl.ds(0, EB)], isb[s], isem).wait()
            pltpu.make_async_copy(dstr.at[pl.ds(0, EB)], idb[ds], isem).wait()

        def issue_gq(bidx, s):
            pltpu.async_copy(tblr.at[isb[s]], gb[s], gsem)
            pltpu.async_copy(qr.at[pl.ds(ebase4(bidx), EB // 4)], qb[s], gsem)

        def wait_gq(s):
            pltpu.make_async_copy(tblr.at[isb[s]], gb[s], gsem).wait()
            pltpu.make_async_copy(qr.at[pl.ds(0, EB // 4)], qb[s], gsem).wait()

        def compute(s):
            def cmp(rq, carry):
                for e in range(4):
                    r = rq * 4 + e
                    for h in (0, 16):
                        v = (gb[s][r, pl.ds(h, 16)]
                             + qb[s][rq, pl.ds(32 * e + h, 16)])
                        mb[s][r, pl.ds(h, 16)] = jnp.maximum(v, 0.1 * v)
                return carry
            lax.fori_loop(0, EB // 4, cmp, 0)

        def scatter(s, ds):
            pltpu.async_copy(mb[s], acc.at[idb[ds]], ssem, add=True)

        def wait_scat(s):
            pltpu.make_async_copy(mb[s], acc.at[id0], ssem).wait()

        def step(b, s, ds, first, last):
            wait_gq(s)
            if not first:
                wait_scat(s)
            if not last:
                load_idx(b + 2, s, (ds + 2) % 4)
            compute(s)
            scatter(s, ds)
            if not last:
                wait_idx(s, (ds + 2) % 4)
                issue_gq(b + 2, s)

        load_idx(0, 0, 0)
        load_idx(1, 1, 1)
        wait_idx(0, 0)
        issue_gq(0, 0)
        wait_idx(1, 1)
        issue_gq(1, 1)
        step(0, 0, 0, True, False)
        step(1, 1, 1, True, False)
        step(2, 0, 2, False, False)
        step(3, 1, 3, False, False)

        def quad(qi, carry):
            b = qi * 4
            step(b + 0, 0, 0, False, False)
            step(b + 1, 1, 1, False, False)
            step(b + 2, 0, 2, False, False)
            step(b + 3, 1, 3, False, False)
            return carry
        lax.fori_loop(1, NBLK // 4 - 1, quad, 0)
        bL = NBLK - 4
        step(bL + 0, 0, 0, False, False)
        step(bL + 1, 1, 1, False, False)
        step(bL + 2, 0, 2, False, True)
        step(bL + 3, 1, 3, False, True)
        wait_scat(0)
        wait_scat(1)
        plsc.subcore_barrier()
        writeback(outr)

    def cnt_pass(dstr, outr):
        fill(m1, 0.0)
        zero_acc(m1)
        fill(m0, 1.0)
        plsc.subcore_barrier()

        def ebase(bidx):
            return pl.multiple_of(t * TPE + bidx * EB, 8)

        def load_idx(bidx, ds):
            pltpu.async_copy(dstr.at[pl.ds(ebase(bidx), EB)], idb[ds], isem)

        def wait_idx(ds):
            pltpu.make_async_copy(dstr.at[pl.ds(0, EB)], idb[ds], isem).wait()

        def wait_scat():
            pltpu.make_async_copy(m0, acc.at[id0], ssem).wait()

        def step(b, ds, first, last):
            if not first:
                wait_scat()
            if not last:
                load_idx(b + 1, (ds + 1) % 4)
            pltpu.async_copy(m0, acc.at[idb[ds]], ssem, add=True)
            if not last:
                wait_idx((ds + 1) % 4)

        load_idx(0, 0)
        wait_idx(0)
        step(0, 0, True, False)
        step(1, 1, True, False)
        step(2, 2, True, False)
        step(3, 3, False, False)

        def quad(qi, carry):
            b = qi * 4
            step(b + 0, 0, False, False)
            step(b + 1, 1, False, False)
            step(b + 2, 2, False, False)
            step(b + 3, 3, False, False)
            return carry
        lax.fori_loop(1, NBLK // 4 - 1, quad, 0)
        bL = NBLK - 4
        step(bL + 0, 0, False, False)
        step(bL + 1, 1, False, False)
        step(bL + 2, 2, False, False)
        step(bL + 3, 3, False, True)
        for _ in range(3):
            wait_scat()
        plsc.subcore_barrier()
        writeback(outr)

    for p in (0, 1):
        chunk = 2 * cid + p
        feat_pass(src_m.at[0], dst_m.at[0], qvc.at[chunk], tvc.at[chunk],
                  hvc.at[chunk])
        feat_pass(src_m.at[1], dst_m.at[1], qcv.at[chunk], tcv.at[chunk],
                  hcv.at[chunk])
    cnt_pass(dst_m.at[cid], cnt_m.at[cid])


def _sc_call(src_m, dst_m, qvc, qcv, tvc, tcv):
    mesh = plsc.VectorSubcoreMesh(core_axis_name="c", subcore_axis_name="s")
    f = pl.kernel(
        _sc_body,
        out_type=[
            jax.ShapeDtypeStruct((NCH, NPAD, 32), jnp.float32),
            jax.ShapeDtypeStruct((NCH, NPAD, 32), jnp.float32),
            jax.ShapeDtypeStruct((2, NPAD, 32), jnp.float32),
        ],
        mesh=mesh,
        compiler_params=pltpu.CompilerParams(use_tc_tiling_on_sc=False),
        scratch_types=(
            [pltpu.VMEM_SHARED((NPAD, 32), jnp.float32)]
            + [pltpu.VMEM((EB,), jnp.int32)] * 6
            + [pltpu.VMEM((EB, 32), jnp.float32)] * 2
            + [pltpu.VMEM((EB // 4, 128), jnp.float32)] * 2
            + [pltpu.VMEM((EB, 32), jnp.float32)] * 2
            + [pltpu.SemaphoreType.DMA] * 4
        ),
    )
    return f(src_m, dst_m, qvc, qcv, tvc, tcv)


def _node_body(feat_ref, h0_ref, h1_ref, h2_ref, h3_ref, cnt_ref, emb_ref,
               gid_ref, ctx_ref, wf_ref, wh_ref, wc_ref, we_ref, b_ref,
               new_ref, agg_ref, gcnt_ref):
    i = pl.program_id(0)
    cnt = jnp.maximum(cnt_ref[0][:, 0:1], 1.0)
    hs = jnp.concatenate(
        [h0_ref[0], h1_ref[0], h2_ref[0], h3_ref[0]], axis=1)
    h = hs / cnt
    dn = (((1,), (0,)), ((), ()))
    x = (jax.lax.dot_general(feat_ref[...], wf_ref[...], dn,
                             preferred_element_type=jnp.float32)
         + jax.lax.dot_general(h, wh_ref[...], dn,
                               preferred_element_type=jnp.float32)
         + jax.lax.dot_general(emb_ref[...], we_ref[...], dn,
                               preferred_element_type=jnp.float32)
         + b_ref[...])
    tctx = jax.lax.dot_general(ctx_ref[...], wc_ref[...], dn,
                               preferred_element_type=jnp.float32)
    gid = gid_ref[0]
    iota = jax.lax.broadcasted_iota(jnp.int32, (NG, gid.shape[1]), 0)
    ohT = (gid == iota).astype(jnp.float32)
    ctx_part = jax.lax.dot_general(ohT, tctx, (((0,), (0,)), ((), ())),
                                   preferred_element_type=jnp.float32)
    new = _leaky(x + ctx_part)
    new_ref[...] = new

    agg = jax.lax.dot_general(ohT, new, (((1,), (0,)), ((), ())),
                              preferred_element_type=jnp.float32)
    gc = jnp.sum(ohT, axis=1, keepdims=True) * jnp.ones((1, EMB), jnp.float32)

    @pl.when(i == 0)
    def _():
        agg_ref[...] = agg
        gcnt_ref[...] = gc

    @pl.when(i != 0)
    def _():
        agg_ref[...] = agg_ref[...] + agg
        gcnt_ref[...] = gcnt_ref[...] + gc


def _node_call(feat, hs, cnt, cnt_idx, emb, gid3d, ctx_emb, wf, wh, wc, we, b):
    B = 2048
    nb = NPAD // B
    hspec = [pl.BlockSpec((1, B, 32), (lambda i, j=j: (j, i, 0)))
             for j in range(NCH)]
    return pl.pallas_call(
        _node_body,
        grid=(nb,),
        in_specs=[
            pl.BlockSpec((B, 32), lambda i: (i, 0)),
            *hspec,
            pl.BlockSpec((1, B, 32), lambda i: (cnt_idx, i, 0)),
            pl.BlockSpec((B, EMB), lambda i: (i, 0)),
            pl.BlockSpec((1, 1, B), lambda i: (i, 0, 0)),
            pl.BlockSpec((NG, EMB), lambda i: (0, 0)),
            pl.BlockSpec((32, EMB), lambda i: (0, 0)),
            pl.BlockSpec((EMB, EMB), lambda i: (0, 0)),
            pl.BlockSpec((EMB, EMB), lambda i: (0, 0)),
            pl.BlockSpec((EMB, EMB), lambda i: (0, 0)),
            pl.BlockSpec((1, EMB), lambda i: (0, 0)),
        ],
        out_specs=[
            pl.BlockSpec((B, EMB), lambda i: (i, 0)),
            pl.BlockSpec((NG, EMB), lambda i: (0, 0)),
            pl.BlockSpec((NG, EMB), lambda i: (0, 0)),
        ],
        out_shape=[
            jax.ShapeDtypeStruct((NPAD, EMB), jnp.float32),
            jax.ShapeDtypeStruct((NG, EMB), jnp.float32),
            jax.ShapeDtypeStruct((NG, EMB), jnp.float32),
        ],
    )(feat, hs, hs, hs, hs, cnt, emb, gid3d, ctx_emb, wf, wh, wc, we, b)


def _ctx_body(uf_ref, cs_ref, cc_ref, vs_ref, vc_ref, ue_ref,
              wuf_ref, wuc_ref, wuv_ref, wue_ref, b_ref, out_ref):
    dn = (((1,), (0,)), ((), ()))
    c_agg = cs_ref[...] / jnp.maximum(cc_ref[...], 1.0)
    v_agg = vs_ref[...] / jnp.maximum(vc_ref[...], 1.0)
    x = (jax.lax.dot_general(uf_ref[...], wuf_ref[...], dn,
                             preferred_element_type=jnp.float32)
         + jax.lax.dot_general(c_agg, wuc_ref[...], dn,
                               preferred_element_type=jnp.float32)
         + jax.lax.dot_general(v_agg, wuv_ref[...], dn,
                               preferred_element_type=jnp.float32)
         + jax.lax.dot_general(ue_ref[...], wue_ref[...], dn,
                               preferred_element_type=jnp.float32)
         + b_ref[...])
    out_ref[...] = _leaky(x)


def _ctx_call(ctx_feat, cs, cc, vs, vc, ctx_emb, wuf, wuc, wuv, wue, b):
    return pl.pallas_call(
        _ctx_body,
        out_shape=jax.ShapeDtypeStruct((NG, EMB), jnp.float32),
    )(ctx_feat, cs, cc, vs, vc, ctx_emb, wuf, wuc, wuv, wue, b)


def kernel(var_feat, clause_feat, ctx_feat, var_emb, clause_emb, ctx_emb,
           edge_vc, edge_sat_vc, edge_cv, edge_sat_cv,
           graph_id_var, graph_id_clause,
           W_mvc, b_mvc, W_mcv, b_mcv, W_cu, b_cu, W_vu, b_vu, W_uu, b_uu):
    f32 = jnp.float32

    def pad_rows(x, n, val=0.0):
        return jnp.concatenate(
            [x, jnp.full((n - x.shape[0],) + x.shape[1:], val, x.dtype)], axis=0)

    ve_p = pad_rows(var_emb.astype(f32), NPAD)
    ce_p = pad_rows(clause_emb.astype(f32), NPAD)
    vf_p = pad_rows(var_feat.astype(f32), NPAD)
    cf_p = pad_rows(clause_feat.astype(f32), NPAD)
    sat_v_p = pad_rows(edge_sat_vc.astype(f32), EPAD)
    sat_c_p = pad_rows(edge_sat_cv.astype(f32), EPAD)

    src_m = jnp.stack([
        pad_rows(edge_vc[0].astype(jnp.int32), EPAD, 0),
        pad_rows(edge_cv[0].astype(jnp.int32), EPAD, 0),
    ])
    dst_m = jnp.stack([
        pad_rows(edge_vc[1].astype(jnp.int32), EPAD, NC),
        pad_rows(edge_cv[1].astype(jnp.int32), EPAD, NV),
    ])

    gid_v = pad_rows(graph_id_var.astype(jnp.int32), NPAD, NG).reshape(NPAD // 2048, 1, 2048)
    gid_c = pad_rows(graph_id_clause.astype(jnp.int32), NPAD, NG).reshape(NPAD // 2048, 1, 2048)

    W_mvc = W_mvc.astype(f32); W_mcv = W_mcv.astype(f32)
    w1v, w2v = W_mvc[:EF], W_mvc[EF:]
    w1c, w2c = W_mcv[:EF], W_mcv[EF:]
    wqv, bqv = _pack_qw(w1v, b_mvc.astype(f32))
    wqc, bqc = _pack_qw(w1c, b_mcv.astype(f32))
    sat4_v = sat_v_p.reshape(EPAD // 4, 4 * EF)
    sat4_c = sat_c_p.reshape(EPAD // 4, 4 * EF)

    W_cu = W_cu.astype(f32); W_vu = W_vu.astype(f32); W_uu = W_uu.astype(f32)
    wcu_f, wcu_h, wcu_x, wcu_e = W_cu[:CF], W_cu[CF:CF + EMB], W_cu[CF + EMB:CF + 2 * EMB], W_cu[CF + 2 * EMB:]
    wvu_f, wvu_h, wvu_x, wvu_e = W_vu[:VF], W_vu[VF:VF + EMB], W_vu[VF + EMB:VF + 2 * EMB], W_vu[VF + 2 * EMB:]
    wuu_f, wuu_c, wuu_v, wuu_e = W_uu[:UF], W_uu[UF:UF + EMB], W_uu[UF + EMB:UF + 2 * EMB], W_uu[UF + 2 * EMB:]
    bcu = b_cu.astype(f32).reshape(1, EMB)
    bvu = b_vu.astype(f32).reshape(1, EMB)
    buu = b_uu.astype(f32).reshape(1, EMB)
    ctx_emb = ctx_emb.astype(f32)
    ctx_feat = ctx_feat.astype(f32)

    qvc, qcv = _q_call(sat4_v, sat4_c, wqv, wqc, bqv, bqc)
    tvc, tcv = _pre_call(ve_p.reshape(NPAD // 4, 4 * EMB),
                         ce_p.reshape(NPAD // 4, 4 * EMB),
                         _pack_tw(w2v), _pack_tw(w2c))

    hvc, hcv, cnt_m = _sc_call(src_m, dst_m, qvc, qcv,
                               tvc.reshape(NCH, NPAD, 32),
                               tcv.reshape(NCH, NPAD, 32))

    new_c_p, c_agg_s, c_gcnt = _node_call(
        cf_p, hvc, cnt_m, 0, ce_p, gid_c, ctx_emb,
        wcu_f, wcu_h, wcu_x, wcu_e, bcu)
    new_v_p, v_agg_s, v_gcnt = _node_call(
        vf_p, hcv, cnt_m, 1, ve_p, gid_v, ctx_emb,
        wvu_f, wvu_h, wvu_x, wvu_e, bvu)

    new_u = _ctx_call(ctx_feat, c_agg_s, c_gcnt, v_agg_s, v_gcnt, ctx_emb,
                      wuu_f, wuu_c, wuu_v, wuu_e, buu)

    return (new_v_p[:NV], new_c_p[:NC], new_u)


# speedup vs baseline: 2.2440x; 1.0316x over previous
import functools
import jax
import jax.numpy as jnp
from jax import lax
from jax.experimental import pallas as pl
from jax.experimental.pallas import tpu as pltpu
from jax.experimental.pallas import tpu_sc as plsc

EMB = 128; CF = 32; VF = 32; UF = 32; EF = 16
NV = 50000; NC = 50000; NG = 64; E = 300000

EPAD = 311296
NPAD = 51200
TPE = EPAD // 16
EB = 128
NBLK = TPE // EB
RB = 128
RPT = NPAD // 16
NCH = 4


def _leaky(x):
    return jnp.where(x > 0, x, 0.1 * x)


def _pre_body(ve_ref, ce_ref, w2v_ref, w2c_ref, tv_ref, tc_ref):
    pv = jax.lax.dot_general(ve_ref[...], w2v_ref[...], (((1,), (0,)), ((), ())),
                             preferred_element_type=jnp.float32)
    pc = jax.lax.dot_general(ce_ref[...], w2c_ref[...], (((1,), (0,)), ((), ())),
                             preferred_element_type=jnp.float32)
    for j in range(NCH):
        tv_ref[j] = pv[:, 128 * j:128 * j + 128]
        tc_ref[j] = pc[:, 128 * j:128 * j + 128]


def _pre_call(ve4, ce4, w2v, w2c):
    nb = NPAD // 2048
    return pl.pallas_call(
        _pre_body,
        grid=(nb,),
        in_specs=[
            pl.BlockSpec((512, 4 * EMB), lambda i: (i, 0)),
            pl.BlockSpec((512, 4 * EMB), lambda i: (i, 0)),
            pl.BlockSpec((4 * EMB, 4 * EMB), lambda i: (0, 0)),
            pl.BlockSpec((4 * EMB, 4 * EMB), lambda i: (0, 0)),
        ],
        out_specs=[pl.BlockSpec((NCH, 512, 128), lambda i: (0, i, 0))] * 2,
        out_shape=[jax.ShapeDtypeStruct((NCH, NPAD // 4, 128), jnp.float32)] * 2,
    )(ve4, ce4, w2v, w2c)


def _pack_tw(w2):
    blocks = w2.reshape(EMB, NCH, 32)
    big = jnp.zeros((4, EMB, NCH, 4, 32), w2.dtype)
    for a in range(4):
        big = big.at[a, :, :, a, :].set(blocks)
    return big.reshape(4 * EMB, 4 * EMB)


def _q_body(sv_ref, sc_ref, wv_ref, wc_ref, bv_ref, bc_ref, qv_ref, qc_ref):
    qv = jax.lax.dot_general(sv_ref[...], wv_ref[...], (((1,), (0,)), ((), ())),
                             preferred_element_type=jnp.float32) + bv_ref[...]
    qc = jax.lax.dot_general(sc_ref[...], wc_ref[...], (((1,), (0,)), ((), ())),
                             preferred_element_type=jnp.float32) + bc_ref[...]
    for j in range(NCH):
        qv_ref[j] = qv[:, 128 * j:128 * j + 128]
        qc_ref[j] = qc[:, 128 * j:128 * j + 128]


def _q_call(sat4_v, sat4_c, wv, wc, bv, bc):
    nb = EPAD // 2048
    return pl.pallas_call(
        _q_body,
        grid=(nb,),
        in_specs=[
            pl.BlockSpec((512, 4 * EF), lambda i: (i, 0)),
            pl.BlockSpec((512, 4 * EF), lambda i: (i, 0)),
            pl.BlockSpec((4 * EF, 4 * EMB), lambda i: (0, 0)),
            pl.BlockSpec((4 * EF, 4 * EMB), lambda i: (0, 0)),
            pl.BlockSpec((1, 4 * EMB), lambda i: (0, 0)),
            pl.BlockSpec((1, 4 * EMB), lambda i: (0, 0)),
        ],
        out_specs=[pl.BlockSpec((NCH, 512, 128), lambda i: (0, i, 0))] * 2,
        out_shape=[jax.ShapeDtypeStruct((NCH, EPAD // 4, 128), jnp.float32)] * 2,
    )(sat4_v, sat4_c, wv, wc, bv, bc)


def _pack_qw(w1, b):
    blocks = w1.reshape(EF, NCH, 32)
    big = jnp.zeros((4, EF, NCH, 4, 32), w1.dtype)
    for a in range(4):
        big = big.at[a, :, :, a, :].set(blocks)
    bbig = jnp.tile(b.reshape(NCH, 1, 32), (1, 4, 1))
    return big.reshape(4 * EF, 4 * EMB), bbig.reshape(1, 4 * EMB)


def _sc_body(src_m, dst_m, qvc, qcv, tvc, tcv,
             hvc, hcv,
             acc,
             is0, is1, id0, id1, id2, id3,
             g0, g1, q0, q1, m0, m1,
             gsem, isem, ssem, wsem):
    cid = lax.axis_index("c")
    t = lax.axis_index("s")
    isb = (is0, is1)
    idb = (id0, id1, id2, id3)
    gb = (g0, g1)
    qb = (q0, q1)
    mb = (m0, m1)

    def fill(buf, val):
        def fi(r, carry):
            buf[r, pl.ds(0, 16)] = jnp.full((16,), val, jnp.float32)
            buf[r, pl.ds(16, 16)] = jnp.full((16,), val, jnp.float32)
            return carry
        lax.fori_loop(0, EB, fi, 0)

    def zero_acc(zbuf):
        def zi(rb, carry):
            r = pl.multiple_of(t * RPT + rb * RB, 8)
            pltpu.async_copy(zbuf, acc.at[pl.ds(r, RB)], wsem)
            return carry
        lax.fori_loop(0, RPT // RB, zi, 0)

        def zw(rb, carry):
            pltpu.make_async_copy(zbuf, acc.at[pl.ds(0, RB)], wsem).wait()
            return carry
        lax.fori_loop(0, RPT // RB, zw, 0)

    def writeback(outr):
        def wi(rb, carry):
            r = pl.multiple_of(t * RPT + rb * RB, 8)
            pltpu.async_copy(acc.at[pl.ds(r, RB)], outr.at[pl.ds(r, RB)], wsem)
            return carry
        lax.fori_loop(0, RPT // RB, wi, 0)

        def ww(rb, carry):
            pltpu.make_async_copy(acc.at[pl.ds(0, RB)], outr.at[pl.ds(0, RB)],
                                  wsem).wait()
            return carry
        lax.fori_loop(0, RPT // RB, ww, 0)

    def feat_pass(srcr, dstr, qr, tblr, outr):
        fill(m0, 0.0)
        zero_acc(m0)
        plsc.subcore_barrier()

        def ebase(bidx):
            return pl.multiple_of(t * TPE + bidx * EB, 8)

        def ebase4(bidx):
            return pl.multiple_of(t * (TPE // 4) + bidx * (EB // 4), 8)

        def load_idx(bidx, s, ds):
            pltpu.async_copy(srcr.at[pl.ds(ebase(bidx), EB)], isb[s], isem)
            pltpu.async_copy(dstr.at[pl.ds(ebase(bidx), EB)], idb[ds], isem)

        def wait_idx(s, ds):
            pltpu.make_async_copy(srcr.at[pl.ds(0, EB)], isb[s], isem).wait()
            pltpu.make_async_copy(dstr.at[pl.ds(0, EB)], idb[ds], isem).wait()

        def issue_gq(bidx, s):
            pltpu.async_copy(tblr.at[isb[s]], gb[s], gsem)
            pltpu.async_copy(qr.at[pl.ds(ebase4(bidx), EB // 4)], qb[s], gsem)

        def wait_gq(s):
            pltpu.make_async_copy(tblr.at[isb[s]], gb[s], gsem).wait()
            pltpu.make_async_copy(qr.at[pl.ds(0, EB // 4)], qb[s], gsem).wait()

        def compute(s):
            def cmp(rq, carry):
                for e in range(4):
                    r = rq * 4 + e
                    for h in (0, 16):
                        v = (gb[s][r, pl.ds(h, 16)]
                             + qb[s][rq, pl.ds(32 * e + h, 16)])
                        mb[s][r, pl.ds(h, 16)] = jnp.maximum(v, 0.1 * v)
                return carry
            lax.fori_loop(0, EB // 4, cmp, 0)

        def scatter(s, ds):
            pltpu.async_copy(mb[s], acc.at[idb[ds]], ssem, add=True)

        def wait_scat(s):
            pltpu.make_async_copy(mb[s], acc.at[id0], ssem).wait()

        def step(b, s, ds, first, last):
            wait_gq(s)
            if not first:
                wait_scat(s)
            if not last:
                load_idx(b + 2, s, (ds + 2) % 4)
            compute(s)
            scatter(s, ds)
            if not last:
                wait_idx(s, (ds + 2) % 4)
                issue_gq(b + 2, s)

        load_idx(0, 0, 0)
        load_idx(1, 1, 1)
        wait_idx(0, 0)
        issue_gq(0, 0)
        wait_idx(1, 1)
        issue_gq(1, 1)
        step(0, 0, 0, True, False)
        step(1, 1, 1, True, False)
        step(2, 0, 2, False, False)
        step(3, 1, 3, False, False)

        def quad(qi, carry):
            b = qi * 4
            step(b + 0, 0, 0, False, False)
            step(b + 1, 1, 1, False, False)
            step(b + 2, 0, 2, False, False)
            step(b + 3, 1, 3, False, False)
            return carry
        lax.fori_loop(1, NBLK // 4 - 1, quad, 0)
        bL = NBLK - 4
        step(bL + 0, 0, 0, False, False)
        step(bL + 1, 1, 1, False, False)
        step(bL + 2, 0, 2, False, True)
        step(bL + 3, 1, 3, False, True)
        wait_scat(0)
        wait_scat(1)
        plsc.subcore_barrier()
        writeback(outr)

    for p in (0, 1):
        chunk = 2 * cid + p
        feat_pass(src_m.at[0], dst_m.at[0], qvc.at[chunk], tvc.at[chunk],
                  hvc.at[chunk])
        feat_pass(src_m.at[1], dst_m.at[1], qcv.at[chunk], tcv.at[chunk],
                  hcv.at[chunk])


def _sc_call(src_m, dst_m, qvc, qcv, tvc, tcv):
    mesh = plsc.VectorSubcoreMesh(core_axis_name="c", subcore_axis_name="s")
    f = pl.kernel(
        _sc_body,
        out_type=[
            jax.ShapeDtypeStruct((NCH, NPAD, 32), jnp.float32),
            jax.ShapeDtypeStruct((NCH, NPAD, 32), jnp.float32),
        ],
        mesh=mesh,
        compiler_params=pltpu.CompilerParams(use_tc_tiling_on_sc=False),
        scratch_types=(
            [pltpu.VMEM_SHARED((NPAD, 32), jnp.float32)]
            + [pltpu.VMEM((EB,), jnp.int32)] * 6
            + [pltpu.VMEM((EB, 32), jnp.float32)] * 2
            + [pltpu.VMEM((EB // 4, 128), jnp.float32)] * 2
            + [pltpu.VMEM((EB, 32), jnp.float32)] * 2
            + [pltpu.SemaphoreType.DMA] * 4
        ),
    )
    return f(src_m, dst_m, qvc, qcv, tvc, tcv)


def _cnt_body(dst_m, cnt_m, acc, id0, id1, id2, id3, m0, m1,
              isem, ssem, wsem):
    cid = lax.axis_index("c")
    t = lax.axis_index("s")
    idb = (id0, id1, id2, id3)

    def fill(buf, val):
        def fi(r, carry):
            buf[r, pl.ds(0, 16)] = jnp.full((16,), val, jnp.float32)
            buf[r, pl.ds(16, 16)] = jnp.full((16,), val, jnp.float32)
            return carry
        lax.fori_loop(0, EB, fi, 0)

    def zero_acc(zbuf):
        def zi(rb, carry):
            r = pl.multiple_of(t * RPT + rb * RB, 8)
            pltpu.async_copy(zbuf, acc.at[pl.ds(r, RB)], wsem)
            return carry
        lax.fori_loop(0, RPT // RB, zi, 0)

        def zw(rb, carry):
            pltpu.make_async_copy(zbuf, acc.at[pl.ds(0, RB)], wsem).wait()
            return carry
        lax.fori_loop(0, RPT // RB, zw, 0)

    def writeback(outr):
        def wi(rb, carry):
            r = pl.multiple_of(t * RPT + rb * RB, 8)
            pltpu.async_copy(acc.at[pl.ds(r, RB)], outr.at[pl.ds(r, RB)], wsem)
            return carry
        lax.fori_loop(0, RPT // RB, wi, 0)

        def ww(rb, carry):
            pltpu.make_async_copy(acc.at[pl.ds(0, RB)], outr.at[pl.ds(0, RB)],
                                  wsem).wait()
            return carry
        lax.fori_loop(0, RPT // RB, ww, 0)

    dstr = dst_m.at[cid]
    outr = cnt_m.at[cid]
    fill(m1, 0.0)
    zero_acc(m1)
    fill(m0, 1.0)
    plsc.subcore_barrier()

    def ebase(bidx):
        return pl.multiple_of(t * TPE + bidx * EB, 8)

    def load_idx(bidx, ds):
        pltpu.async_copy(dstr.at[pl.ds(ebase(bidx), EB)], idb[ds], isem)

    def wait_idx(ds):
        pltpu.make_async_copy(dstr.at[pl.ds(0, EB)], idb[ds], isem).wait()

    def wait_scat():
        pltpu.make_async_copy(m0, acc.at[id0], ssem).wait()

    def step(b, ds, first, last):
        if not first:
            wait_scat()
        if not last:
            load_idx(b + 1, (ds + 1) % 4)
        pltpu.async_copy(m0, acc.at[idb[ds]], ssem, add=True)
        if not last:
            wait_idx((ds + 1) % 4)

    load_idx(0, 0)
    wait_idx(0)
    step(0, 0, True, False)
    step(1, 1, True, False)
    step(2, 2, True, False)
    step(3, 3, False, False)

    def quad(qi, carry):
        b = qi * 4
        step(b + 0, 0, False, False)
        step(b + 1, 1, False, False)
        step(b + 2, 2, False, False)
        step(b + 3, 3, False, False)
        return carry
    lax.fori_loop(1, NBLK // 4 - 1, quad, 0)
    bL = NBLK - 4
    step(bL + 0, 0, False, False)
    step(bL + 1, 1, False, False)
    step(bL + 2, 2, False, False)
    step(bL + 3, 3, False, True)
    for _ in range(3):
        wait_scat()
    plsc.subcore_barrier()
    writeback(outr)


def _cnt_call(dst_m):
    mesh = plsc.VectorSubcoreMesh(core_axis_name="c", subcore_axis_name="s")
    f = pl.kernel(
        _cnt_body,
        out_type=[jax.ShapeDtypeStruct((2, NPAD, 32), jnp.float32)],
        mesh=mesh,
        compiler_params=pltpu.CompilerParams(use_tc_tiling_on_sc=False),
        scratch_types=(
            [pltpu.VMEM_SHARED((NPAD, 32), jnp.float32)]
            + [pltpu.VMEM((EB,), jnp.int32)] * 4
            + [pltpu.VMEM((EB, 32), jnp.float32)] * 2
            + [pltpu.SemaphoreType.DMA] * 3
        ),
    )
    return f(dst_m)[0]


def _node_body(feat_ref, h0_ref, h1_ref, h2_ref, h3_ref, cnt_ref, emb_ref,
               gid_ref, ctx_ref, wf_ref, wh_ref, wc_ref, we_ref, b_ref,
               new_ref, agg_ref, gcnt_ref):
    i = pl.program_id(0)
    cnt = jnp.maximum(cnt_ref[0][:, 0:1], 1.0)
    hs = jnp.concatenate(
        [h0_ref[0], h1_ref[0], h2_ref[0], h3_ref[0]], axis=1)
    h = hs / cnt
    dn = (((1,), (0,)), ((), ()))
    x = (jax.lax.dot_general(feat_ref[...], wf_ref[...], dn,
                             preferred_element_type=jnp.float32)
         + jax.lax.dot_general(h, wh_ref[...], dn,
                               preferred_element_type=jnp.float32)
         + jax.lax.dot_general(emb_ref[...], we_ref[...], dn,
                               preferred_element_type=jnp.float32)
         + b_ref[...])
    tctx = jax.lax.dot_general(ctx_ref[...], wc_ref[...], dn,
                               preferred_element_type=jnp.float32)
    gid = gid_ref[0]
    iota = jax.lax.broadcasted_iota(jnp.int32, (NG, gid.shape[1]), 0)
    ohT = (gid == iota).astype(jnp.float32)
    ctx_part = jax.lax.dot_general(ohT, tctx, (((0,), (0,)), ((), ())),
                                   preferred_element_type=jnp.float32)
    new = _leaky(x + ctx_part)
    new_ref[...] = new

    agg = jax.lax.dot_general(ohT, new, (((1,), (0,)), ((), ())),
                              preferred_element_type=jnp.float32)
    gc = jnp.sum(ohT, axis=1, keepdims=True) * jnp.ones((1, EMB), jnp.float32)

    @pl.when(i == 0)
    def _():
        agg_ref[...] = agg
        gcnt_ref[...] = gc

    @pl.when(i != 0)
    def _():
        agg_ref[...] = agg_ref[...] + agg
        gcnt_ref[...] = gcnt_ref[...] + gc


def _node_call(feat, hs, cnt, cnt_idx, emb, gid3d, ctx_emb, wf, wh, wc, we, b):
    B = 2048
    nb = NPAD // B
    hspec = [pl.BlockSpec((1, B, 32), (lambda i, j=j: (j, i, 0)))
             for j in range(NCH)]
    return pl.pallas_call(
        _node_body,
        grid=(nb,),
        in_specs=[
            pl.BlockSpec((B, 32), lambda i: (i, 0)),
            *hspec,
            pl.BlockSpec((1, B, 32), lambda i: (cnt_idx, i, 0)),
            pl.BlockSpec((B, EMB), lambda i: (i, 0)),
            pl.BlockSpec((1, 1, B), lambda i: (i, 0, 0)),
            pl.BlockSpec((NG, EMB), lambda i: (0, 0)),
            pl.BlockSpec((32, EMB), lambda i: (0, 0)),
            pl.BlockSpec((EMB, EMB), lambda i: (0, 0)),
            pl.BlockSpec((EMB, EMB), lambda i: (0, 0)),
            pl.BlockSpec((EMB, EMB), lambda i: (0, 0)),
            pl.BlockSpec((1, EMB), lambda i: (0, 0)),
        ],
        out_specs=[
            pl.BlockSpec((B, EMB), lambda i: (i, 0)),
            pl.BlockSpec((NG, EMB), lambda i: (0, 0)),
            pl.BlockSpec((NG, EMB), lambda i: (0, 0)),
        ],
        out_shape=[
            jax.ShapeDtypeStruct((NPAD, EMB), jnp.float32),
            jax.ShapeDtypeStruct((NG, EMB), jnp.float32),
            jax.ShapeDtypeStruct((NG, EMB), jnp.float32),
        ],
    )(feat, hs, hs, hs, hs, cnt, emb, gid3d, ctx_emb, wf, wh, wc, we, b)


def _ctx_body(uf_ref, cs_ref, cc_ref, vs_ref, vc_ref, ue_ref,
              wuf_ref, wuc_ref, wuv_ref, wue_ref, b_ref, out_ref):
    dn = (((1,), (0,)), ((), ()))
    c_agg = cs_ref[...] / jnp.maximum(cc_ref[...], 1.0)
    v_agg = vs_ref[...] / jnp.maximum(vc_ref[...], 1.0)
    x = (jax.lax.dot_general(uf_ref[...], wuf_ref[...], dn,
                             preferred_element_type=jnp.float32)
         + jax.lax.dot_general(c_agg, wuc_ref[...], dn,
                               preferred_element_type=jnp.float32)
         + jax.lax.dot_general(v_agg, wuv_ref[...], dn,
                               preferred_element_type=jnp.float32)
         + jax.lax.dot_general(ue_ref[...], wue_ref[...], dn,
                               preferred_element_type=jnp.float32)
         + b_ref[...])
    out_ref[...] = _leaky(x)


def _ctx_call(ctx_feat, cs, cc, vs, vc, ctx_emb, wuf, wuc, wuv, wue, b):
    return pl.pallas_call(
        _ctx_body,
        out_shape=jax.ShapeDtypeStruct((NG, EMB), jnp.float32),
    )(ctx_feat, cs, cc, vs, vc, ctx_emb, wuf, wuc, wuv, wue, b)


def kernel(var_feat, clause_feat, ctx_feat, var_emb, clause_emb, ctx_emb,
           edge_vc, edge_sat_vc, edge_cv, edge_sat_cv,
           graph_id_var, graph_id_clause,
           W_mvc, b_mvc, W_mcv, b_mcv, W_cu, b_cu, W_vu, b_vu, W_uu, b_uu):
    f32 = jnp.float32

    def pad_rows(x, n, val=0.0):
        return jnp.concatenate(
            [x, jnp.full((n - x.shape[0],) + x.shape[1:], val, x.dtype)], axis=0)

    ve_p = pad_rows(var_emb.astype(f32), NPAD)
    ce_p = pad_rows(clause_emb.astype(f32), NPAD)
    vf_p = pad_rows(var_feat.astype(f32), NPAD)
    cf_p = pad_rows(clause_feat.astype(f32), NPAD)
    sat_v_p = pad_rows(edge_sat_vc.astype(f32), EPAD)
    sat_c_p = pad_rows(edge_sat_cv.astype(f32), EPAD)

    src_m = jnp.stack([
        pad_rows(edge_vc[0].astype(jnp.int32), EPAD, 0),
        pad_rows(edge_cv[0].astype(jnp.int32), EPAD, 0),
    ])
    dst_m = jnp.stack([
        pad_rows(edge_vc[1].astype(jnp.int32), EPAD, NC),
        pad_rows(edge_cv[1].astype(jnp.int32), EPAD, NV),
    ])

    gid_v = pad_rows(graph_id_var.astype(jnp.int32), NPAD, NG).reshape(NPAD // 2048, 1, 2048)
    gid_c = pad_rows(graph_id_clause.astype(jnp.int32), NPAD, NG).reshape(NPAD // 2048, 1, 2048)

    W_mvc = W_mvc.astype(f32); W_mcv = W_mcv.astype(f32)
    w1v, w2v = W_mvc[:EF], W_mvc[EF:]
    w1c, w2c = W_mcv[:EF], W_mcv[EF:]
    wqv, bqv = _pack_qw(w1v, b_mvc.astype(f32))
    wqc, bqc = _pack_qw(w1c, b_mcv.astype(f32))
    sat4_v = sat_v_p.reshape(EPAD // 4, 4 * EF)
    sat4_c = sat_c_p.reshape(EPAD // 4, 4 * EF)

    W_cu = W_cu.astype(f32); W_vu = W_vu.astype(f32); W_uu = W_uu.astype(f32)
    wcu_f, wcu_h, wcu_x, wcu_e = W_cu[:CF], W_cu[CF:CF + EMB], W_cu[CF + EMB:CF + 2 * EMB], W_cu[CF + 2 * EMB:]
    wvu_f, wvu_h, wvu_x, wvu_e = W_vu[:VF], W_vu[VF:VF + EMB], W_vu[VF + EMB:VF + 2 * EMB], W_vu[VF + 2 * EMB:]
    wuu_f, wuu_c, wuu_v, wuu_e = W_uu[:UF], W_uu[UF:UF + EMB], W_uu[UF + EMB:UF + 2 * EMB], W_uu[UF + 2 * EMB:]
    bcu = b_cu.astype(f32).reshape(1, EMB)
    bvu = b_vu.astype(f32).reshape(1, EMB)
    buu = b_uu.astype(f32).reshape(1, EMB)
    ctx_emb = ctx_emb.astype(f32)
    ctx_feat = ctx_feat.astype(f32)

    cnt_m = _cnt_call(dst_m)

    qvc, qcv = _q_call(sat4_v, sat4_c, wqv, wqc, bqv, bqc)
    tvc, tcv = _pre_call(ve_p.reshape(NPAD // 4, 4 * EMB),
                         ce_p.reshape(NPAD // 4, 4 * EMB),
                         _pack_tw(w2v), _pack_tw(w2c))

    hvc, hcv = _sc_call(src_m, dst_m, qvc, qcv,
                        tvc.reshape(NCH, NPAD, 32),
                        tcv.reshape(NCH, NPAD, 32))

    new_c_p, c_agg_s, c_gcnt = _node_call(
        cf_p, hvc, cnt_m, 0, ce_p, gid_c, ctx_emb,
        wcu_f, wcu_h, wcu_x, wcu_e, bcu)
    new_v_p, v_agg_s, v_gcnt = _node_call(
        vf_p, hcv, cnt_m, 1, ve_p, gid_v, ctx_emb,
        wvu_f, wvu_h, wvu_x, wvu_e, bvu)

    new_u = _ctx_call(ctx_feat, c_agg_s, c_gcnt, v_agg_s, v_gcnt, ctx_emb,
                      wuu_f, wuu_c, wuu_v, wuu_e, buu)

    return (new_v_p[:NV], new_c_p[:NC], new_u)
